# Initial kernel scaffold; baseline (speedup 1.0000x reference)
import jax
import jax.numpy as jnp
from jax.experimental import pallas as pl


def kernel(x_src, pos_src, x_dst, pos_dst, edge_index, edge_attr, params):
    raise NotImplementedError("write your pallas kernel here")


# speedup vs baseline: 2.6746x; 2.6746x over previous
import functools
import math

import jax
import jax.numpy as jnp
from jax import lax
from jax.experimental import pallas as pl
from jax.experimental.pallas import tpu as pltpu
from jax.experimental.pallas import tpu_sc as plsc

N = 10000
E = 320000
F = 128
WG = 256
EDGE_DIM = 16
CUTOFF = 10.0
EPS = 1e-6

BN = 2000
BE = 2560

NC = 2
NS = 16
NW = NC * NS
PER_T = E // NW
CH = 80
NCH = PER_T // CH

SCH = 128
TOT_CHUNKS = E // SCH
TPS_CHUNKS = TOT_CHUNKS // NS
TPS_EXTRA = TOT_CHUNKS - NS * TPS_CHUNKS


def _leaky(x):
    return jnp.where(x >= 0, x, 0.01 * x)


def _elu(x):
    return jnp.where(x > 0, x, jnp.exp(jnp.minimum(x, 0.0)) - 1.0)


def _vn_leaky(w00, x):
    d = w00 * x
    dot = jnp.sum(x * d, axis=1, keepdims=True)
    msk = (dot >= 0).astype(x.dtype)
    dsq = jnp.sum(d * d, axis=1, keepdims=True)
    return 0.01 * x + 0.99 * (msk * x + (1.0 - msk) * (x - (dot / (dsq + EPS)) * d))


def _gvl(sca, vec, Ws_ref, wsv_ref, bs_ref, gw_ref, c5):
    lvw, lvb, l2w, l2b, gb = c5
    vi = lvw * vec + lvb
    vn = jnp.sqrt(jnp.sum(vi * vi, axis=1, keepdims=True) + 1e-12)
    so = (jnp.dot(sca, Ws_ref[...], preferred_element_type=jnp.float32)
          + vn * wsv_ref[...] + bs_ref[...])
    gate = jax.nn.sigmoid(jnp.sum(so * gw_ref[...], axis=1, keepdims=True) + gb)
    return so, gate * (l2w * vi + l2b)


def _pack_gvl(p, in_s):
    w = p['lin_scalar']['w']
    return (jnp.transpose(w[:, :in_s]),
            jnp.transpose(w[:, in_s:in_s + 1]),
            p['lin_scalar']['b'][None, :],
            p['gates']['w'])


def _gvl_consts(p):
    return [p['lin_vector']['w'][0, 0], p['lin_vector']['b'][0],
            p['lin_vector2']['w'][0, 0], p['lin_vector2']['b'][0],
            p['gates']['b'][0]]


def _node_pre_body(consts, xs_ref, ps_ref, xd_ref, pd_ref,
                   s1Ws, s1wv, s1b, s1g, d1Ws, d1wv, d1b, d1g,
                   ngWs, ngwv, ngb, ngg, ctWs, ctwv, ctb, ctg,
                   Ts_ref, Td_ref, cs_ref, cv_ref):
    def c5(i):
        return tuple(consts[i + k] for k in range(5))

    for side in range(2):
        x = (xs_ref if side == 0 else xd_ref)[...]
        p3 = (ps_ref if side == 0 else pd_ref)[:, 0:3]
        pk = (s1Ws, s1wv, s1b, s1g) if side == 0 else (d1Ws, d1wv, d1b, d1g)
        so, vo = _gvl(x, p3, *pk, c5(0 if side == 0 else 5))
        xn = _leaky(so)
        pv = _vn_leaky(consts[10 + side], vo)
        ns, nv = _gvl(xn, pv, ngWs, ngwv, ngb, ngg, c5(12))
        T_ref = Ts_ref if side == 0 else Td_ref
        T_ref[:, 0:F] = ns
        T_ref[:, F:F + 3] = nv
        T_ref[:, F + 3:F + 6] = p3
        T_ref[:, F + 6:WG] = jnp.zeros((BN, WG - F - 6), jnp.float32)
        if side == 0:
            cs, cv = _gvl(xn, pv, ctWs, ctwv, ctb, ctg, c5(17))
            cs_ref[...] = cs
            cv_ref[:, 0:3] = cv
            cv_ref[:, 3:8] = jnp.zeros((BN, 5), jnp.float32)


def _node_pre(x_src, pos_src8, x_dst, pos_dst8, consts, wlist):
    full = lambda a: pl.BlockSpec(a.shape, lambda i: (0,) * a.ndim)
    blk = lambda s: pl.BlockSpec(s, lambda i: (i,) + (0,) * (len(s) - 1))
    return pl.pallas_call(
        _node_pre_body,
        grid=(N // BN,),
        in_specs=[pl.BlockSpec(memory_space=pltpu.SMEM),
                  blk((BN, F)), blk((BN, 8)), blk((BN, F)), blk((BN, 8))]
                 + [full(a) for a in wlist],
        out_specs=[blk((BN, WG)), blk((BN, WG)), blk((BN, F)), blk((BN, 8))],
        out_shape=[jax.ShapeDtypeStruct((N, WG), jnp.float32),
                   jax.ShapeDtypeStruct((N, WG), jnp.float32),
                   jax.ShapeDtypeStruct((N, F), jnp.float32),
                   jax.ShapeDtypeStruct((N, 8), jnp.float32)],
    )(consts, x_src, pos_src8, x_dst, pos_dst8, *wlist)


def _sc_gather(td, ts, col, row):
    mesh = plsc.VectorSubcoreMesh(core_axis_name="c", subcore_axis_name="s",
                                  num_cores=NC, num_subcores=NS)

    @functools.partial(
        pl.kernel, mesh=mesh,
        out_type=[jax.ShapeDtypeStruct((E, WG), jnp.float32),
                  jax.ShapeDtypeStruct((E, WG), jnp.float32)],
        scratch_types=[pltpu.VMEM((CH,), jnp.int32),
                       pltpu.VMEM((CH, WG), jnp.float32),
                       pltpu.VMEM((CH,), jnp.int32),
                       pltpu.VMEM((CH, WG), jnp.float32),
                       pltpu.SemaphoreType.DMA,
                       pltpu.SemaphoreType.DMA],
    )
    def gather_k(td_h, ts_h, col_h, row_h, gd_h, gs_h,
                 idx_d, rows_d, idx_s, rows_s, sem_d, sem_s):
        wid = lax.axis_index("s") * NC + lax.axis_index("c")
        base = wid * PER_T

        def body(j, _):
            off = base + j * CH
            pltpu.sync_copy(col_h.at[pl.ds(off, CH)], idx_d)
            pltpu.sync_copy(row_h.at[pl.ds(off, CH)], idx_s)
            cd = pltpu.async_copy(td_h.at[idx_d], rows_d, sem_d)
            cs_ = pltpu.async_copy(ts_h.at[idx_s], rows_s, sem_s)
            cd.wait()
            pltpu.sync_copy(rows_d, gd_h.at[pl.ds(off, CH)])
            cs_.wait()
            pltpu.sync_copy(rows_s, gs_h.at[pl.ds(off, CH)])
            return 0

        lax.fori_loop(0, NCH, body, 0)

    return gather_k(td, ts, col, row)


def _edge_body(consts, gd_ref, gs_ref, ea_ref,
               Wsm, Wea, wev, be, gew, Wt, bt, we2n, wn2e,
               Wo, wov, bo, gow, m_ref, mv_ref):
    (elvw, elvb, el2w, el2b, geb, vn_e, vexw, evnw, be2n, bn2e,
     olvw, olvb, ol2w, ol2b, gob) = (consts[k] for k in range(15))

    pd = gd_ref[:, F + 3:F + 6]
    ps = gs_ref[:, F + 3:F + 6]
    ev = ps - pd
    d2 = jnp.sum(ev * ev, axis=1, keepdims=True)
    ed = jnp.sqrt(d2 + 1e-12)

    off = (lax.broadcasted_iota(jnp.int32, (1, EDGE_DIM), 1).astype(jnp.float32)
           * (CUTOFF / (EDGE_DIM - 1)))
    dd = ed - off
    sm = jnp.exp((-0.5 * (EDGE_DIM - 1) * (EDGE_DIM - 1) / (CUTOFF * CUTOFF)) * dd * dd)

    e_vec = (ev / (ed + 1e-7)) * vexw

    vi_e = elvw * e_vec + elvb
    vne = jnp.sqrt(jnp.sum(vi_e * vi_e, axis=1, keepdims=True) + 1e-12)
    es0 = (jnp.dot(sm, Wsm[...], preferred_element_type=jnp.float32)
           + jnp.dot(ea_ref[...], Wea[...], preferred_element_type=jnp.float32)
           + vne * wev[...] + be[...])
    gate_e = jax.nn.sigmoid(jnp.sum(es0 * gew[...], axis=1, keepdims=True) + geb)
    ve = gate_e * (el2w * vi_e + el2b)
    es = _leaky(es0)
    evg = _vn_leaky(vn_e, ve)

    t = jnp.dot(es, Wt[...], preferred_element_type=jnp.float32) + bt[...]
    c1 = jnp.sum(es * we2n[...], axis=1, keepdims=True) + be2n
    ev2 = evnw * evg

    C = 0.5 * (jnp.cos(ed * (math.pi / CUTOFF)) + 1.0)
    C = C * (ed <= CUTOFF).astype(jnp.float32) * (ed >= 0.0).astype(jnp.float32)

    def msg(ns, nv):
        y_sca = ns * t
        c2 = jnp.sum(ns * wn2e[...], axis=1, keepdims=True) + bn2e
        y_v = c1 * nv + c2 * ev2
        vi = olvw * y_v + olvb
        vno = jnp.sqrt(jnp.sum(vi * vi, axis=1, keepdims=True) + 1e-12)
        os_ = (jnp.dot(y_sca, Wo[...], preferred_element_type=jnp.float32)
               + vno * wov[...] + bo[...])
        gate = jax.nn.sigmoid(jnp.sum(os_ * gow[...], axis=1, keepdims=True) + gob)
        ov = gate * (ol2w * vi + ol2b)
        return os_ * C, ov * C

    os1, ov1 = msg(gd_ref[:, 0:F], gd_ref[:, F:F + 3])
    os2, ov2 = msg(gs_ref[:, 0:F], gs_ref[:, F:F + 3])
    m_ref[...] = _elu((os1 + os2) * 0.5)
    mv_ref[:, 0:3] = _elu((ov1 + ov2) * 0.5)
    mv_ref[:, 3:F] = jnp.zeros((BE, F - 3), jnp.float32)


def _edge_stage(gd, gs, ea, consts, wlist):
    full = lambda a: pl.BlockSpec(a.shape, lambda i: (0,) * a.ndim)
    blk = lambda s: pl.BlockSpec(s, lambda i: (i,) + (0,) * (len(s) - 1))
    return pl.pallas_call(
        _edge_body,
        grid=(E // BE,),
        in_specs=[pl.BlockSpec(memory_space=pltpu.SMEM),
                  blk((BE, WG)), blk((BE, WG)), blk((BE, EDGE_DIM))]
                 + [full(a) for a in wlist],
        out_specs=[blk((BE, F)), blk((BE, F))],
        out_shape=[jax.ShapeDtypeStruct((E, F), jnp.float32),
                   jax.ShapeDtypeStruct((E, F), jnp.float32)],
    )(consts, gd, gs, ea, *wlist)


def _sc_scatter(m, mv, row):
    mesh = plsc.VectorSubcoreMesh(core_axis_name="c", subcore_axis_name="s",
                                  num_cores=NC, num_subcores=NS)

    @functools.partial(
        pl.kernel, mesh=mesh,
        out_type=jax.ShapeDtypeStruct((NC, N, F), jnp.float32),
        scratch_types=[pltpu.VMEM((SCH,), jnp.int32),
                       pltpu.VMEM((SCH, F), jnp.float32),
                       pltpu.VMEM((8, F), jnp.float32),
                       pltpu.VMEM_SHARED((N, F), jnp.float32)],
    )
    def scatter_k(m_h, mv_h, row_h, p_h, idx_v, mbuf, zbuf, acc):
        c = lax.axis_index("c")
        s = lax.axis_index("s")

        zv16 = jnp.zeros((16,), jnp.float32)

        def zrow(i, _):
            for k in range(F // 16):
                zbuf[i, pl.ds(16 * k, 16)] = zv16
            return 0

        lax.fori_loop(0, 8, zrow, 0)

        nrc = 78 + jnp.where(s < 2, 1, 0)
        rbase = s * 624 + 8 * jnp.minimum(s, 2)

        def za(j, _):
            pltpu.sync_copy(zbuf, acc.at[pl.ds(rbase + 8 * j, 8)])
            return 0

        lax.fori_loop(0, nrc, za, 0)
        plsc.subcore_barrier()

        nch = TPS_CHUNKS + jnp.where(s < TPS_EXTRA, 1, 0)
        cbase = s * TPS_CHUNKS + jnp.minimum(s, TPS_EXTRA)

        def mkloop(src_ref):
            def body(j, _):
                off = (cbase + j) * SCH
                pltpu.sync_copy(row_h.at[pl.ds(off, SCH)], idx_v)
                pltpu.sync_copy(src_ref.at[pl.ds(off, SCH)], mbuf)
                pltpu.sync_copy(mbuf, acc.at[idx_v], add=True)
                return 0
            return body

        @pl.when(c == 0)
        def _():
            lax.fori_loop(0, nch, mkloop(m_h), 0)

        @pl.when(c == 1)
        def _():
            lax.fori_loop(0, nch, mkloop(mv_h), 0)

        plsc.subcore_barrier()

        def wb(j, _):
            r = rbase + 8 * j
            pltpu.sync_copy(acc.at[pl.ds(r, 8)], zbuf)
            pltpu.sync_copy(zbuf, p_h.at[c, pl.ds(r, 8)])
            return 0

        lax.fori_loop(0, nrc, wb, 0)

    return scatter_k(m, mv, row)


def _post_body(consts, p_ref, cs_ref, cv_ref,
               lng, lnb, lvg, lvb_, tWs, twv, tb, tg, out_s_ref, out_v_ref):
    tlvw, tlvb, tl2w, tl2b, tgb, actw = (consts[k] for k in range(6))
    s = cs_ref[...] + p_ref[0, :, :]
    v = cv_ref[:, 0:3] + p_ref[1, :, 0:3]
    m = jnp.mean(s, axis=1, keepdims=True)
    va = jnp.mean((s - m) * (s - m), axis=1, keepdims=True)
    s = (s - m) / jnp.sqrt(va + 1e-5) * lng[...] + lnb[...]
    mv = jnp.mean(v, axis=1, keepdims=True)
    vv = jnp.mean((v - mv) * (v - mv), axis=1, keepdims=True)
    v = (v - mv) / jnp.sqrt(vv + 1e-5) * lvg[:, 0:3] + lvb_[:, 0:3]
    s = _leaky(s)
    v = _vn_leaky(actw, v)
    so, vo = _gvl(s, v, tWs, twv, tb, tg, (tlvw, tlvb, tl2w, tl2b, tgb))
    out_s_ref[...] = so
    out_v_ref[:, 0:3] = vo
    out_v_ref[:, 3:8] = jnp.zeros((BN, 5), jnp.float32)


def _node_post(p, cs, cv8, consts, wlist):
    full = lambda a: pl.BlockSpec(a.shape, lambda i: (0,) * a.ndim)
    blk = lambda s: pl.BlockSpec(s, lambda i: (i,) + (0,) * (len(s) - 1))
    return pl.pallas_call(
        _post_body,
        grid=(N // BN,),
        in_specs=[pl.BlockSpec(memory_space=pltpu.SMEM),
                  pl.BlockSpec((NC, BN, F), lambda i: (0, i, 0)),
                  blk((BN, F)), blk((BN, 8))]
                 + [full(a) for a in wlist],
        out_specs=[blk((BN, F)), blk((BN, 8))],
        out_shape=[jax.ShapeDtypeStruct((N, F), jnp.float32),
                   jax.ShapeDtypeStruct((N, 8), jnp.float32)],
    )(consts, p, cs, cv8, *wlist)


def kernel(x_src, pos_src, x_dst, pos_dst, edge_index, edge_attr, params):
    f32 = jnp.float32
    row = edge_index[0]
    col = edge_index[1]
    pos_src8 = jnp.pad(pos_src.astype(f32), ((0, 0), (0, 5)))
    pos_dst8 = jnp.pad(pos_dst.astype(f32), ((0, 0), (0, 5)))

    p1s, p1d = params['per1_src'], params['per1_dst']
    msg1 = params['msg1']
    c1 = jnp.stack(
        _gvl_consts(p1s['gv']) + _gvl_consts(p1d['gv'])
        + [p1s['vn_dir'][0, 0], p1d['vn_dir'][0, 0]]
        + _gvl_consts(msg1['node_gv']) + _gvl_consts(params['centroid']))
    w1 = (list(_pack_gvl(p1s['gv'], F)) + list(_pack_gvl(p1d['gv'], F))
          + list(_pack_gvl(msg1['node_gv'], F)) + list(_pack_gvl(params['centroid'], F)))
    ts, td, cs, cv8 = _node_pre(x_src, pos_src8, x_dst, pos_dst8, c1, w1)

    gd, gs = _sc_gather(td, ts, col, row)

    eg = msg1['edge_gvp']
    egWs, egwv, egb, egg = _pack_gvl(eg['gv'], 2 * EDGE_DIM)
    ec = _gvl_consts(eg['gv'])
    oWs, owv, ob, og = _pack_gvl(msg1['out_gv'], F)
    oc = _gvl_consts(msg1['out_gv'])
    c3 = jnp.stack(ec + [eg['vn_dir'][0, 0], params['vec_exp_w'][0, 0],
                         msg1['edge_vn']['w'][0, 0], msg1['e2n']['b'][0],
                         msg1['n2e']['b'][0]] + oc)
    w3 = [egWs[:EDGE_DIM], egWs[EDGE_DIM:], egwv, egb, egg,
          jnp.transpose(msg1['sca_linear']['w']), msg1['sca_linear']['b'][None, :],
          msg1['e2n']['w'], msg1['n2e']['w'],
          oWs, owv, ob, og]
    m, mv = _edge_stage(gd, gs, edge_attr.astype(f32), c3, w3)

    p = _sc_scatter(m, mv, row)

    ot = params['out_transform']
    c5 = jnp.stack(_gvl_consts(ot) + [params['act_vec_w'][0, 0]])
    w5 = [params['ln_sca']['g'][None, :], params['ln_sca']['b'][None, :],
          jnp.pad(params['ln_vec']['g'], ((0, 0), (0, 5))),
          jnp.pad(params['ln_vec']['b'], ((0, 0), (0, 5)))] + list(_pack_gvl(ot, F))
    out_s, out_v8 = _node_post(p, cs, cv8, c5, w5)

    return out_s, out_v8[:, :3].reshape(N, 1, 3)


# speedup vs baseline: 3.0503x; 1.1405x over previous
import functools
import math

import jax
import jax.numpy as jnp
from jax import lax
from jax.experimental import pallas as pl
from jax.experimental.pallas import tpu as pltpu
from jax.experimental.pallas import tpu_sc as plsc

N = 10000
E = 320000
F = 128
WG = 256
EDGE_DIM = 16
CUTOFF = 10.0
EPS = 1e-6

BN = 2000
BE = 2560

NC = 2
NS = 16
NW = NC * NS
PER_T = E // NW
CH = 80
NCH = PER_T // CH

SCH = 128
TOT_CHUNKS = E // SCH
TPS_CHUNKS = TOT_CHUNKS // NS
TPS_EXTRA = TOT_CHUNKS - NS * TPS_CHUNKS


def _leaky(x):
    return jnp.where(x >= 0, x, 0.01 * x)


def _elu(x):
    return jnp.where(x > 0, x, jnp.exp(jnp.minimum(x, 0.0)) - 1.0)


def _vn_leaky(w00, x):
    d = w00 * x
    dot = jnp.sum(x * d, axis=1, keepdims=True)
    msk = (dot >= 0).astype(x.dtype)
    dsq = jnp.sum(d * d, axis=1, keepdims=True)
    return 0.01 * x + 0.99 * (msk * x + (1.0 - msk) * (x - (dot / (dsq + EPS)) * d))


def _gvl(sca, vec, Ws_ref, wsv_ref, bs_ref, gw_ref, c5):
    lvw, lvb, l2w, l2b, gb = c5
    vi = lvw * vec + lvb
    vn = jnp.sqrt(jnp.sum(vi * vi, axis=1, keepdims=True) + 1e-12)
    so = (jnp.dot(sca, Ws_ref[...], preferred_element_type=jnp.float32)
          + vn * wsv_ref[...] + bs_ref[...])
    gate = jax.nn.sigmoid(
        jnp.dot(so, gw_ref[...], preferred_element_type=jnp.float32) + gb)
    return so, gate * (l2w * vi + l2b)


def _pack_gvl(p, in_s):
    w = p['lin_scalar']['w']
    return (jnp.transpose(w[:, :in_s]),
            jnp.transpose(w[:, in_s:in_s + 1]),
            p['lin_scalar']['b'][None, :],
            jnp.transpose(p['gates']['w']))


def _gvl_consts(p):
    return [p['lin_vector']['w'][0, 0], p['lin_vector']['b'][0],
            p['lin_vector2']['w'][0, 0], p['lin_vector2']['b'][0],
            p['gates']['b'][0]]


def _node_pre_body(consts, xs_ref, ps_ref, xd_ref, pd_ref,
                   s1Ws, s1wv, s1b, s1g, d1Ws, d1wv, d1b, d1g,
                   ngWs, ngwv, ngb, ngg, ctWs, ctwv, ctb, ctg,
                   Ts_ref, Td_ref, cs_ref, cv_ref):
    def c5(i):
        return tuple(consts[i + k] for k in range(5))

    for side in range(2):
        x = (xs_ref if side == 0 else xd_ref)[...]
        p3 = (ps_ref if side == 0 else pd_ref)[:, 0:3]
        pk = (s1Ws, s1wv, s1b, s1g) if side == 0 else (d1Ws, d1wv, d1b, d1g)
        so, vo = _gvl(x, p3, *pk, c5(0 if side == 0 else 5))
        xn = _leaky(so)
        pv = _vn_leaky(consts[10 + side], vo)
        ns, nv = _gvl(xn, pv, ngWs, ngwv, ngb, ngg, c5(12))
        T_ref = Ts_ref if side == 0 else Td_ref
        T_ref[:, 0:F] = ns
        T_ref[:, F:F + 3] = nv
        T_ref[:, F + 3:F + 6] = p3
        T_ref[:, F + 6:WG] = jnp.zeros((BN, WG - F - 6), jnp.float32)
        if side == 0:
            cs, cv = _gvl(xn, pv, ctWs, ctwv, ctb, ctg, c5(17))
            cs_ref[...] = cs
            cv_ref[:, 0:3] = cv
            cv_ref[:, 3:8] = jnp.zeros((BN, 5), jnp.float32)


def _node_pre(x_src, pos_src8, x_dst, pos_dst8, consts, wlist):
    full = lambda a: pl.BlockSpec(a.shape, lambda i: (0,) * a.ndim)
    blk = lambda s: pl.BlockSpec(s, lambda i: (i,) + (0,) * (len(s) - 1))
    return pl.pallas_call(
        _node_pre_body,
        grid=(N // BN,),
        in_specs=[pl.BlockSpec(memory_space=pltpu.SMEM),
                  blk((BN, F)), blk((BN, 8)), blk((BN, F)), blk((BN, 8))]
                 + [full(a) for a in wlist],
        out_specs=[blk((BN, WG)), blk((BN, WG)), blk((BN, F)), blk((BN, 8))],
        out_shape=[jax.ShapeDtypeStruct((N, WG), jnp.float32),
                   jax.ShapeDtypeStruct((N, WG), jnp.float32),
                   jax.ShapeDtypeStruct((N, F), jnp.float32),
                   jax.ShapeDtypeStruct((N, 8), jnp.float32)],
    )(consts, x_src, pos_src8, x_dst, pos_dst8, *wlist)


def _sc_gather(td, ts, col, row):
    mesh = plsc.VectorSubcoreMesh(core_axis_name="c", subcore_axis_name="s",
                                  num_cores=NC, num_subcores=NS)

    @functools.partial(
        pl.kernel, mesh=mesh,
        out_type=[jax.ShapeDtypeStruct((E, WG), jnp.float32),
                  jax.ShapeDtypeStruct((E, WG), jnp.float32)],
        scratch_types=[pltpu.VMEM((CH,), jnp.int32),
                       pltpu.VMEM((CH, WG), jnp.float32),
                       pltpu.VMEM((CH,), jnp.int32),
                       pltpu.VMEM((CH, WG), jnp.float32),
                       pltpu.SemaphoreType.DMA,
                       pltpu.SemaphoreType.DMA],
    )
    def gather_k(td_h, ts_h, col_h, row_h, gd_h, gs_h,
                 idx_d, rows_d, idx_s, rows_s, sem_d, sem_s):
        wid = lax.axis_index("s") * NC + lax.axis_index("c")
        base = wid * PER_T

        def body(j, _):
            off = base + j * CH
            pltpu.sync_copy(col_h.at[pl.ds(off, CH)], idx_d)
            pltpu.sync_copy(row_h.at[pl.ds(off, CH)], idx_s)
            cd = pltpu.async_copy(td_h.at[idx_d], rows_d, sem_d)
            cs_ = pltpu.async_copy(ts_h.at[idx_s], rows_s, sem_s)
            cd.wait()
            pltpu.sync_copy(rows_d, gd_h.at[pl.ds(off, CH)])
            cs_.wait()
            pltpu.sync_copy(rows_s, gs_h.at[pl.ds(off, CH)])
            return 0

        lax.fori_loop(0, NCH, body, 0)

    return gather_k(td, ts, col, row)


def _edge_body(consts, gd_ref, gs_ref, ea_ref,
               Wsm, Wea, wev, be, gew, Wt, bt, we2n, wn2e,
               Wo, wov, bo, gow, m_ref, mv_ref):
    (elvw, elvb, el2w, el2b, geb, vn_e, vexw, evnw, be2n, bn2e,
     olvw, olvb, ol2w, ol2b, gob) = (consts[k] for k in range(15))

    pd = gd_ref[:, F + 3:F + 6]
    ps = gs_ref[:, F + 3:F + 6]
    ev = ps - pd
    d2 = jnp.sum(ev * ev, axis=1, keepdims=True)
    ed = jnp.sqrt(d2 + 1e-12)

    off = (lax.broadcasted_iota(jnp.int32, (1, EDGE_DIM), 1).astype(jnp.float32)
           * (CUTOFF / (EDGE_DIM - 1)))
    dd = ed - off
    sm = jnp.exp((-0.5 * (EDGE_DIM - 1) * (EDGE_DIM - 1) / (CUTOFF * CUTOFF)) * dd * dd)

    e_vec = ev * ((1.0 / (ed + 1e-7)) * vexw)

    vi_e = elvw * e_vec + elvb
    vne = jnp.sqrt(jnp.sum(vi_e * vi_e, axis=1, keepdims=True) + 1e-12)
    es0 = (jnp.dot(sm, Wsm[...], preferred_element_type=jnp.float32)
           + jnp.dot(ea_ref[...], Wea[...], preferred_element_type=jnp.float32)
           + vne * wev[...] + be[...])
    gate_e = jax.nn.sigmoid(
        jnp.dot(es0, gew[...], preferred_element_type=jnp.float32) + geb)
    ve = gate_e * (el2w * vi_e + el2b)
    es = _leaky(es0)
    evg = _vn_leaky(vn_e, ve)

    t = jnp.dot(es, Wt[...], preferred_element_type=jnp.float32) + bt[...]
    c1 = jnp.dot(es, we2n[...], preferred_element_type=jnp.float32) + be2n
    ev2 = evnw * evg

    u = d2 * ((math.pi / CUTOFF) ** 2)
    fact = [1.0]
    for k in range(1, 13):
        fact.append(fact[-1] * (2 * k - 1) * (2 * k))
    poly = ((-1.0) ** 12) / fact[12]
    for k in range(11, -1, -1):
        poly = poly * u + ((-1.0) ** k) / fact[k]
    C = 0.5 * (1.0 + poly)
    C = C * (ed <= CUTOFF).astype(jnp.float32) * (ed >= 0.0).astype(jnp.float32)

    def msg(ns, nv):
        y_sca = ns * t
        c2 = jnp.dot(ns, wn2e[...], preferred_element_type=jnp.float32) + bn2e
        y_v = c1 * nv + c2 * ev2
        vi = olvw * y_v + olvb
        vno = jnp.sqrt(jnp.sum(vi * vi, axis=1, keepdims=True) + 1e-12)
        os_ = (jnp.dot(y_sca, Wo[...], preferred_element_type=jnp.float32)
               + vno * wov[...] + bo[...])
        gate = jax.nn.sigmoid(
            jnp.dot(os_, gow[...], preferred_element_type=jnp.float32) + gob)
        ov = gate * (ol2w * vi + ol2b)
        return os_ * C, ov * C

    os1, ov1 = msg(gd_ref[:, 0:F], gd_ref[:, F:F + 3])
    os2, ov2 = msg(gs_ref[:, 0:F], gs_ref[:, F:F + 3])
    m_ref[...] = _elu((os1 + os2) * 0.5)
    mv_ref[:, 0:3] = _elu((ov1 + ov2) * 0.5)
    mv_ref[:, 3:F] = jnp.zeros((BE, F - 3), jnp.float32)


def _edge_stage(gd, gs, ea, consts, wlist):
    full = lambda a: pl.BlockSpec(a.shape, lambda i: (0,) * a.ndim)
    blk = lambda s: pl.BlockSpec(s, lambda i: (i,) + (0,) * (len(s) - 1))
    return pl.pallas_call(
        _edge_body,
        grid=(E // BE,),
        in_specs=[pl.BlockSpec(memory_space=pltpu.SMEM),
                  blk((BE, WG)), blk((BE, WG)), blk((BE, EDGE_DIM))]
                 + [full(a) for a in wlist],
        out_specs=[blk((BE, F)), blk((BE, F))],
        out_shape=[jax.ShapeDtypeStruct((E, F), jnp.float32),
                   jax.ShapeDtypeStruct((E, F), jnp.float32)],
    )(consts, gd, gs, ea, *wlist)


def _sc_scatter(m, mv, row):
    mesh = plsc.VectorSubcoreMesh(core_axis_name="c", subcore_axis_name="s",
                                  num_cores=NC, num_subcores=NS)

    @functools.partial(
        pl.kernel, mesh=mesh,
        out_type=jax.ShapeDtypeStruct((NC, N, F), jnp.float32),
        scratch_types=[pltpu.VMEM((SCH,), jnp.int32),
                       pltpu.VMEM((SCH, F), jnp.float32),
                       pltpu.VMEM((8, F), jnp.float32),
                       pltpu.VMEM_SHARED((N, F), jnp.float32)],
    )
    def scatter_k(m_h, mv_h, row_h, p_h, idx_v, mbuf, zbuf, acc):
        c = lax.axis_index("c")
        s = lax.axis_index("s")

        zv16 = jnp.zeros((16,), jnp.float32)

        def zrow(i, _):
            for k in range(F // 16):
                zbuf[i, pl.ds(16 * k, 16)] = zv16
            return 0

        lax.fori_loop(0, 8, zrow, 0)

        nrc = 78 + jnp.where(s < 2, 1, 0)
        rbase = s * 624 + 8 * jnp.minimum(s, 2)

        def za(j, _):
            pltpu.sync_copy(zbuf, acc.at[pl.ds(rbase + 8 * j, 8)])
            return 0

        lax.fori_loop(0, nrc, za, 0)
        plsc.subcore_barrier()

        nch = TPS_CHUNKS + jnp.where(s < TPS_EXTRA, 1, 0)
        cbase = s * TPS_CHUNKS + jnp.minimum(s, TPS_EXTRA)

        def mkloop(src_ref):
            def body(j, _):
                off = (cbase + j) * SCH
                pltpu.sync_copy(row_h.at[pl.ds(off, SCH)], idx_v)
                pltpu.sync_copy(src_ref.at[pl.ds(off, SCH)], mbuf)
                pltpu.sync_copy(mbuf, acc.at[idx_v], add=True)
                return 0
            return body

        @pl.when(c == 0)
        def _():
            lax.fori_loop(0, nch, mkloop(m_h), 0)

        @pl.when(c == 1)
        def _():
            lax.fori_loop(0, nch, mkloop(mv_h), 0)

        plsc.subcore_barrier()

        def wb(j, _):
            r = rbase + 8 * j
            pltpu.sync_copy(acc.at[pl.ds(r, 8)], zbuf)
            pltpu.sync_copy(zbuf, p_h.at[c, pl.ds(r, 8)])
            return 0

        lax.fori_loop(0, nrc, wb, 0)

    return scatter_k(m, mv, row)


def _post_body(consts, p_ref, cs_ref, cv_ref,
               lng, lnb, lvg, lvb_, tWs, twv, tb, tg, out_s_ref, out_v_ref):
    tlvw, tlvb, tl2w, tl2b, tgb, actw = (consts[k] for k in range(6))
    s = cs_ref[...] + p_ref[0, :, :]
    v = cv_ref[:, 0:3] + p_ref[1, :, 0:3]
    m = jnp.mean(s, axis=1, keepdims=True)
    va = jnp.mean((s - m) * (s - m), axis=1, keepdims=True)
    s = (s - m) / jnp.sqrt(va + 1e-5) * lng[...] + lnb[...]
    mv = jnp.mean(v, axis=1, keepdims=True)
    vv = jnp.mean((v - mv) * (v - mv), axis=1, keepdims=True)
    v = (v - mv) / jnp.sqrt(vv + 1e-5) * lvg[:, 0:3] + lvb_[:, 0:3]
    s = _leaky(s)
    v = _vn_leaky(actw, v)
    so, vo = _gvl(s, v, tWs, twv, tb, tg, (tlvw, tlvb, tl2w, tl2b, tgb))
    out_s_ref[...] = so
    out_v_ref[:, 0:3] = vo
    out_v_ref[:, 3:8] = jnp.zeros((BN, 5), jnp.float32)


def _node_post(p, cs, cv8, consts, wlist):
    full = lambda a: pl.BlockSpec(a.shape, lambda i: (0,) * a.ndim)
    blk = lambda s: pl.BlockSpec(s, lambda i: (i,) + (0,) * (len(s) - 1))
    return pl.pallas_call(
        _post_body,
        grid=(N // BN,),
        in_specs=[pl.BlockSpec(memory_space=pltpu.SMEM),
                  pl.BlockSpec((NC, BN, F), lambda i: (0, i, 0)),
                  blk((BN, F)), blk((BN, 8))]
                 + [full(a) for a in wlist],
        out_specs=[blk((BN, F)), blk((BN, 8))],
        out_shape=[jax.ShapeDtypeStruct((N, F), jnp.float32),
                   jax.ShapeDtypeStruct((N, 8), jnp.float32)],
    )(consts, p, cs, cv8, *wlist)


def kernel(x_src, pos_src, x_dst, pos_dst, edge_index, edge_attr, params):
    f32 = jnp.float32
    row = edge_index[0]
    col = edge_index[1]
    pos_src8 = jnp.pad(pos_src.astype(f32), ((0, 0), (0, 5)))
    pos_dst8 = jnp.pad(pos_dst.astype(f32), ((0, 0), (0, 5)))

    p1s, p1d = params['per1_src'], params['per1_dst']
    msg1 = params['msg1']
    c1 = jnp.stack(
        _gvl_consts(p1s['gv']) + _gvl_consts(p1d['gv'])
        + [p1s['vn_dir'][0, 0], p1d['vn_dir'][0, 0]]
        + _gvl_consts(msg1['node_gv']) + _gvl_consts(params['centroid']))
    w1 = (list(_pack_gvl(p1s['gv'], F)) + list(_pack_gvl(p1d['gv'], F))
          + list(_pack_gvl(msg1['node_gv'], F)) + list(_pack_gvl(params['centroid'], F)))
    ts, td, cs, cv8 = _node_pre(x_src, pos_src8, x_dst, pos_dst8, c1, w1)

    gd, gs = _sc_gather(td, ts, col, row)

    eg = msg1['edge_gvp']
    egWs, egwv, egb, egg = _pack_gvl(eg['gv'], 2 * EDGE_DIM)
    ec = _gvl_consts(eg['gv'])
    oWs, owv, ob, og = _pack_gvl(msg1['out_gv'], F)
    oc = _gvl_consts(msg1['out_gv'])
    c3 = jnp.stack(ec + [eg['vn_dir'][0, 0], params['vec_exp_w'][0, 0],
                         msg1['edge_vn']['w'][0, 0], msg1['e2n']['b'][0],
                         msg1['n2e']['b'][0]] + oc)
    w3 = [egWs[:EDGE_DIM], egWs[EDGE_DIM:], egwv, egb, egg,
          jnp.transpose(msg1['sca_linear']['w']), msg1['sca_linear']['b'][None, :],
          jnp.transpose(msg1['e2n']['w']), jnp.transpose(msg1['n2e']['w']),
          oWs, owv, ob, og]
    m, mv = _edge_stage(gd, gs, edge_attr.astype(f32), c3, w3)

    p = _sc_scatter(m, mv, row)

    ot = params['out_transform']
    c5 = jnp.stack(_gvl_consts(ot) + [params['act_vec_w'][0, 0]])
    w5 = [params['ln_sca']['g'][None, :], params['ln_sca']['b'][None, :],
          jnp.pad(params['ln_vec']['g'], ((0, 0), (0, 5))),
          jnp.pad(params['ln_vec']['b'], ((0, 0), (0, 5)))] + list(_pack_gvl(ot, F))
    out_s, out_v8 = _node_post(p, cs, cv8, c5, w5)

    return out_s, out_v8[:, :3].reshape(N, 1, 3)


# speedup vs baseline: 3.3911x; 1.1117x over previous
import functools
import math

import jax
import jax.numpy as jnp
from jax import lax
from jax.experimental import pallas as pl
from jax.experimental.pallas import tpu as pltpu
from jax.experimental.pallas import tpu_sc as plsc

N = 10000
E = 320000
F = 128
WG = 256
EDGE_DIM = 16
CUTOFF = 10.0
EPS = 1e-6

BN = 2000
BE = 2560

NC = 2
NS = 16
NW = NC * NS
PER_T = E // NW
CH = 80
NCH = PER_T // CH

SCH = 128
TOT_CHUNKS = E // SCH
TPS_CHUNKS = TOT_CHUNKS // NS
TPS_EXTRA = TOT_CHUNKS - NS * TPS_CHUNKS


def _leaky(x):
    return jnp.where(x >= 0, x, 0.01 * x)


def _elu(x):
    return jnp.where(x > 0, x, jnp.exp(jnp.minimum(x, 0.0)) - 1.0)


def _vn_leaky(w00, x):
    d = w00 * x
    dot = jnp.sum(x * d, axis=1, keepdims=True)
    msk = (dot >= 0).astype(x.dtype)
    dsq = jnp.sum(d * d, axis=1, keepdims=True)
    return 0.01 * x + 0.99 * (msk * x + (1.0 - msk) * (x - (dot / (dsq + EPS)) * d))


def _gvl(sca, vec, Ws_ref, wsv_ref, bs_ref, gw_ref, c5):
    lvw, lvb, l2w, l2b, gb = c5
    vi = lvw * vec + lvb
    vn = jnp.sqrt(jnp.sum(vi * vi, axis=1, keepdims=True) + 1e-12)
    so = (jnp.dot(sca, Ws_ref[...], preferred_element_type=jnp.float32)
          + vn * wsv_ref[...] + bs_ref[...])
    gate = jax.nn.sigmoid(
        jnp.dot(so, gw_ref[...], preferred_element_type=jnp.float32) + gb)
    return so, gate * (l2w * vi + l2b)


def _pack_gvl(p, in_s):
    w = p['lin_scalar']['w']
    return (jnp.transpose(w[:, :in_s]),
            jnp.transpose(w[:, in_s:in_s + 1]),
            p['lin_scalar']['b'][None, :],
            jnp.transpose(p['gates']['w']))


def _gvl_consts(p):
    return [p['lin_vector']['w'][0, 0], p['lin_vector']['b'][0],
            p['lin_vector2']['w'][0, 0], p['lin_vector2']['b'][0],
            p['gates']['b'][0]]


def _node_pre_body(consts, xs_ref, ps_ref, xd_ref, pd_ref,
                   s1Ws, s1wv, s1b, s1g, d1Ws, d1wv, d1b, d1g,
                   ngWs, ngwv, ngb, ngg, ctWs, ctwv, ctb, ctg,
                   Ts_ref, Td_ref, cs_ref, cv_ref):
    def c5(i):
        return tuple(consts[i + k] for k in range(5))

    for side in range(2):
        x = (xs_ref if side == 0 else xd_ref)[...]
        p3 = (ps_ref if side == 0 else pd_ref)[:, 0:3]
        pk = (s1Ws, s1wv, s1b, s1g) if side == 0 else (d1Ws, d1wv, d1b, d1g)
        so, vo = _gvl(x, p3, *pk, c5(0 if side == 0 else 5))
        xn = _leaky(so)
        pv = _vn_leaky(consts[10 + side], vo)
        ns, nv = _gvl(xn, pv, ngWs, ngwv, ngb, ngg, c5(12))
        T_ref = Ts_ref if side == 0 else Td_ref
        T_ref[:, 0:F] = ns
        T_ref[:, F:F + 3] = nv
        T_ref[:, F + 3:F + 6] = p3
        T_ref[:, F + 6:WG] = jnp.zeros((BN, WG - F - 6), jnp.float32)
        if side == 0:
            cs, cv = _gvl(xn, pv, ctWs, ctwv, ctb, ctg, c5(17))
            cs_ref[...] = cs
            cv_ref[:, 0:3] = cv
            cv_ref[:, 3:8] = jnp.zeros((BN, 5), jnp.float32)


def _node_pre(x_src, pos_src8, x_dst, pos_dst8, consts, wlist):
    full = lambda a: pl.BlockSpec(a.shape, lambda i: (0,) * a.ndim)
    blk = lambda s: pl.BlockSpec(s, lambda i: (i,) + (0,) * (len(s) - 1))
    return pl.pallas_call(
        _node_pre_body,
        grid=(N // BN,),
        in_specs=[pl.BlockSpec(memory_space=pltpu.SMEM),
                  blk((BN, F)), blk((BN, 8)), blk((BN, F)), blk((BN, 8))]
                 + [full(a) for a in wlist],
        out_specs=[blk((BN, WG)), blk((BN, WG)), blk((BN, F)), blk((BN, 8))],
        out_shape=[jax.ShapeDtypeStruct((N, WG), jnp.float32),
                   jax.ShapeDtypeStruct((N, WG), jnp.float32),
                   jax.ShapeDtypeStruct((N, F), jnp.float32),
                   jax.ShapeDtypeStruct((N, 8), jnp.float32)],
    )(consts, x_src, pos_src8, x_dst, pos_dst8, *wlist)


def _sc_gather(td, ts, col, row):
    mesh = plsc.VectorSubcoreMesh(core_axis_name="c", subcore_axis_name="s",
                                  num_cores=NC, num_subcores=NS)

    @functools.partial(
        pl.kernel, mesh=mesh,
        out_type=[jax.ShapeDtypeStruct((E, WG), jnp.float32),
                  jax.ShapeDtypeStruct((E, WG), jnp.float32)],
        scratch_types=[pltpu.VMEM((PER_T,), jnp.int32),
                       pltpu.VMEM((PER_T,), jnp.int32),
                       pltpu.VMEM((CH, WG), jnp.float32),
                       pltpu.VMEM((CH, WG), jnp.float32),
                       pltpu.VMEM((CH, WG), jnp.float32),
                       pltpu.VMEM((CH, WG), jnp.float32),
                       pltpu.SemaphoreType.DMA,
                       pltpu.SemaphoreType.DMA,
                       pltpu.SemaphoreType.DMA,
                       pltpu.SemaphoreType.DMA],
    )
    def gather_k(td_h, ts_h, col_h, row_h, gd_h, gs_h,
                 idx_d, idx_s, rows_d0, rows_s0, rows_d1, rows_s1,
                 sem_d0, sem_s0, sem_d1, sem_s1):
        wid = lax.axis_index("s") * NC + lax.axis_index("c")
        base = wid * PER_T

        pltpu.sync_copy(col_h.at[pl.ds(base, PER_T)], idx_d)
        pltpu.sync_copy(row_h.at[pl.ds(base, PER_T)], idx_s)

        def pair(j2, _):
            j0 = 2 * j2
            j1 = j0 + 1
            off0 = base + j0 * CH
            off1 = base + j1 * CH
            g0d = pltpu.async_copy(
                td_h.at[idx_d.at[pl.ds(j0 * CH, CH)]], rows_d0, sem_d0)
            g0s = pltpu.async_copy(
                ts_h.at[idx_s.at[pl.ds(j0 * CH, CH)]], rows_s0, sem_s0)
            g1d = pltpu.async_copy(
                td_h.at[idx_d.at[pl.ds(j1 * CH, CH)]], rows_d1, sem_d1)
            g1s = pltpu.async_copy(
                ts_h.at[idx_s.at[pl.ds(j1 * CH, CH)]], rows_s1, sem_s1)
            g0d.wait()
            pltpu.sync_copy(rows_d0, gd_h.at[pl.ds(off0, CH)])
            g0s.wait()
            pltpu.sync_copy(rows_s0, gs_h.at[pl.ds(off0, CH)])
            g1d.wait()
            pltpu.sync_copy(rows_d1, gd_h.at[pl.ds(off1, CH)])
            g1s.wait()
            pltpu.sync_copy(rows_s1, gs_h.at[pl.ds(off1, CH)])
            return 0

        lax.fori_loop(0, NCH // 2, pair, 0)

        j = NCH - 1
        off = base + j * CH
        gd_t = pltpu.async_copy(
            td_h.at[idx_d.at[pl.ds(j * CH, CH)]], rows_d0, sem_d0)
        gs_t = pltpu.async_copy(
            ts_h.at[idx_s.at[pl.ds(j * CH, CH)]], rows_s0, sem_s0)
        gd_t.wait()
        pltpu.sync_copy(rows_d0, gd_h.at[pl.ds(off, CH)])
        gs_t.wait()
        pltpu.sync_copy(rows_s0, gs_h.at[pl.ds(off, CH)])

    return gather_k(td, ts, col, row)


def _edge_body(consts, gd_ref, gs_ref, ea_ref,
               Wsm, Wea, wev, be, gew, Wt, bt, we2n, wn2e,
               Wo, wov, bo, gow, m_ref, mv_ref):
    (elvw, elvb, el2w, el2b, geb, vn_e, vexw, evnw, be2n, bn2e,
     olvw, olvb, ol2w, ol2b, gob) = (consts[k] for k in range(15))

    pd = gd_ref[:, F + 3:F + 6]
    ps = gs_ref[:, F + 3:F + 6]
    ev = ps - pd
    d2 = jnp.sum(ev * ev, axis=1, keepdims=True)
    ed = jnp.sqrt(d2 + 1e-12)

    off = (lax.broadcasted_iota(jnp.int32, (1, EDGE_DIM), 1).astype(jnp.float32)
           * (CUTOFF / (EDGE_DIM - 1)))
    dd = ed - off
    sm = jnp.exp((-0.5 * (EDGE_DIM - 1) * (EDGE_DIM - 1) / (CUTOFF * CUTOFF)) * dd * dd)

    e_vec = ev * ((1.0 / (ed + 1e-7)) * vexw)

    vi_e = elvw * e_vec + elvb
    vne = jnp.sqrt(jnp.sum(vi_e * vi_e, axis=1, keepdims=True) + 1e-12)
    es0 = (jnp.dot(sm, Wsm[...], preferred_element_type=jnp.float32)
           + jnp.dot(ea_ref[...], Wea[...], preferred_element_type=jnp.float32)
           + vne * wev[...] + be[...])
    gate_e = jax.nn.sigmoid(
        jnp.dot(es0, gew[...], preferred_element_type=jnp.float32) + geb)
    ve = gate_e * (el2w * vi_e + el2b)
    es = _leaky(es0)
    evg = _vn_leaky(vn_e, ve)

    t = jnp.dot(es, Wt[...], preferred_element_type=jnp.float32) + bt[...]
    c1 = jnp.dot(es, we2n[...], preferred_element_type=jnp.float32) + be2n
    ev2 = evnw * evg

    u = d2 * ((math.pi / CUTOFF) ** 2)
    fact = [1.0]
    for k in range(1, 13):
        fact.append(fact[-1] * (2 * k - 1) * (2 * k))
    poly = ((-1.0) ** 12) / fact[12]
    for k in range(11, -1, -1):
        poly = poly * u + ((-1.0) ** k) / fact[k]
    C = 0.5 * (1.0 + poly)
    C = C * (ed <= CUTOFF).astype(jnp.float32) * (ed >= 0.0).astype(jnp.float32)

    def msg(ns, nv):
        y_sca = ns * t
        c2 = jnp.dot(ns, wn2e[...], preferred_element_type=jnp.float32) + bn2e
        y_v = c1 * nv + c2 * ev2
        vi = olvw * y_v + olvb
        vno = jnp.sqrt(jnp.sum(vi * vi, axis=1, keepdims=True) + 1e-12)
        os_ = (jnp.dot(y_sca, Wo[...], preferred_element_type=jnp.float32)
               + vno * wov[...] + bo[...])
        gate = jax.nn.sigmoid(
            jnp.dot(os_, gow[...], preferred_element_type=jnp.float32) + gob)
        ov = gate * (ol2w * vi + ol2b)
        return os_ * C, ov * C

    os1, ov1 = msg(gd_ref[:, 0:F], gd_ref[:, F:F + 3])
    os2, ov2 = msg(gs_ref[:, 0:F], gs_ref[:, F:F + 3])
    m_ref[...] = _elu((os1 + os2) * 0.5)
    mv_ref[:, 0:3] = _elu((ov1 + ov2) * 0.5)
    mv_ref[:, 3:F] = jnp.zeros((BE, F - 3), jnp.float32)


def _edge_stage(gd, gs, ea, consts, wlist):
    full = lambda a: pl.BlockSpec(a.shape, lambda i: (0,) * a.ndim)
    blk = lambda s: pl.BlockSpec(s, lambda i: (i,) + (0,) * (len(s) - 1))
    return pl.pallas_call(
        _edge_body,
        grid=(E // BE,),
        in_specs=[pl.BlockSpec(memory_space=pltpu.SMEM),
                  blk((BE, WG)), blk((BE, WG)), blk((BE, EDGE_DIM))]
                 + [full(a) for a in wlist],
        out_specs=[blk((BE, F)), blk((BE, F))],
        out_shape=[jax.ShapeDtypeStruct((E, F), jnp.float32),
                   jax.ShapeDtypeStruct((E, F), jnp.float32)],
    )(consts, gd, gs, ea, *wlist)


def _sc_scatter(m, mv, row):
    mesh = plsc.VectorSubcoreMesh(core_axis_name="c", subcore_axis_name="s",
                                  num_cores=NC, num_subcores=NS)

    @functools.partial(
        pl.kernel, mesh=mesh,
        out_type=jax.ShapeDtypeStruct((NC, N, F), jnp.float32),
        scratch_types=[pltpu.VMEM((SCH,), jnp.int32),
                       pltpu.VMEM((SCH,), jnp.int32),
                       pltpu.VMEM((SCH, F), jnp.float32),
                       pltpu.VMEM((SCH, F), jnp.float32),
                       pltpu.VMEM((8, F), jnp.float32),
                       pltpu.VMEM_SHARED((N, F), jnp.float32),
                       pltpu.SemaphoreType.DMA,
                       pltpu.SemaphoreType.DMA],
    )
    def scatter_k(m_h, mv_h, row_h, p_h,
                  idx_v0, idx_v1, mbuf0, mbuf1, zbuf, acc, sml0, sml1):
        c = lax.axis_index("c")
        s = lax.axis_index("s")

        zv16 = jnp.zeros((16,), jnp.float32)

        def zrow(i, _):
            for k in range(F // 16):
                zbuf[i, pl.ds(16 * k, 16)] = zv16
            return 0

        lax.fori_loop(0, 8, zrow, 0)

        nrc = 78 + jnp.where(s < 2, 1, 0)
        rbase = s * 624 + 8 * jnp.minimum(s, 2)

        def za(j, _):
            pltpu.sync_copy(zbuf, acc.at[pl.ds(rbase + 8 * j, 8)])
            return 0

        lax.fori_loop(0, nrc, za, 0)
        plsc.subcore_barrier()

        nch = TPS_CHUNKS + jnp.where(s < TPS_EXTRA, 1, 0)
        cbase = s * TPS_CHUNKS + jnp.minimum(s, TPS_EXTRA)

        def mkloop(src_ref):
            def pair(j2, _):
                j0 = 2 * j2
                j1 = j0 + 1
                j1c = jnp.minimum(j1, nch - 1)
                off0 = (cbase + j0) * SCH
                off1 = (cbase + j1c) * SCH
                l0 = pltpu.async_copy(src_ref.at[pl.ds(off0, SCH)], mbuf0, sml0)
                l1 = pltpu.async_copy(src_ref.at[pl.ds(off1, SCH)], mbuf1, sml1)
                pltpu.sync_copy(row_h.at[pl.ds(off0, SCH)], idx_v0)
                pltpu.sync_copy(row_h.at[pl.ds(off1, SCH)], idx_v1)
                l0.wait()
                pltpu.sync_copy(mbuf0, acc.at[idx_v0], add=True)
                l1.wait()

                @pl.when(j1 < nch)
                def _():
                    pltpu.sync_copy(mbuf1, acc.at[idx_v1], add=True)

                return 0
            return pair

        npairs = (nch + 1) // 2

        @pl.when(c == 0)
        def _():
            lax.fori_loop(0, npairs, mkloop(m_h), 0)

        @pl.when(c == 1)
        def _():
            lax.fori_loop(0, npairs, mkloop(mv_h), 0)

        plsc.subcore_barrier()

        def wb(j, _):
            r = rbase + 8 * j
            pltpu.sync_copy(acc.at[pl.ds(r, 8)], zbuf)
            pltpu.sync_copy(zbuf, p_h.at[c, pl.ds(r, 8)])
            return 0

        lax.fori_loop(0, nrc, wb, 0)

    return scatter_k(m, mv, row)


def _post_body(consts, p_ref, cs_ref, cv_ref,
               lng, lnb, lvg, lvb_, tWs, twv, tb, tg, out_s_ref, out_v_ref):
    tlvw, tlvb, tl2w, tl2b, tgb, actw = (consts[k] for k in range(6))
    s = cs_ref[...] + p_ref[0, :, :]
    v = cv_ref[:, 0:3] + p_ref[1, :, 0:3]
    m = jnp.mean(s, axis=1, keepdims=True)
    va = jnp.mean((s - m) * (s - m), axis=1, keepdims=True)
    s = (s - m) / jnp.sqrt(va + 1e-5) * lng[...] + lnb[...]
    mv = jnp.mean(v, axis=1, keepdims=True)
    vv = jnp.mean((v - mv) * (v - mv), axis=1, keepdims=True)
    v = (v - mv) / jnp.sqrt(vv + 1e-5) * lvg[:, 0:3] + lvb_[:, 0:3]
    s = _leaky(s)
    v = _vn_leaky(actw, v)
    so, vo = _gvl(s, v, tWs, twv, tb, tg, (tlvw, tlvb, tl2w, tl2b, tgb))
    out_s_ref[...] = so
    out_v_ref[:, 0:3] = vo
    out_v_ref[:, 3:8] = jnp.zeros((BN, 5), jnp.float32)


def _node_post(p, cs, cv8, consts, wlist):
    full = lambda a: pl.BlockSpec(a.shape, lambda i: (0,) * a.ndim)
    blk = lambda s: pl.BlockSpec(s, lambda i: (i,) + (0,) * (len(s) - 1))
    return pl.pallas_call(
        _post_body,
        grid=(N // BN,),
        in_specs=[pl.BlockSpec(memory_space=pltpu.SMEM),
                  pl.BlockSpec((NC, BN, F), lambda i: (0, i, 0)),
                  blk((BN, F)), blk((BN, 8))]
                 + [full(a) for a in wlist],
        out_specs=[blk((BN, F)), blk((BN, 8))],
        out_shape=[jax.ShapeDtypeStruct((N, F), jnp.float32),
                   jax.ShapeDtypeStruct((N, 8), jnp.float32)],
    )(consts, p, cs, cv8, *wlist)


def kernel(x_src, pos_src, x_dst, pos_dst, edge_index, edge_attr, params):
    f32 = jnp.float32
    row = edge_index[0]
    col = edge_index[1]
    pos_src8 = jnp.pad(pos_src.astype(f32), ((0, 0), (0, 5)))
    pos_dst8 = jnp.pad(pos_dst.astype(f32), ((0, 0), (0, 5)))

    p1s, p1d = params['per1_src'], params['per1_dst']
    msg1 = params['msg1']
    c1 = jnp.stack(
        _gvl_consts(p1s['gv']) + _gvl_consts(p1d['gv'])
        + [p1s['vn_dir'][0, 0], p1d['vn_dir'][0, 0]]
        + _gvl_consts(msg1['node_gv']) + _gvl_consts(params['centroid']))
    w1 = (list(_pack_gvl(p1s['gv'], F)) + list(_pack_gvl(p1d['gv'], F))
          + list(_pack_gvl(msg1['node_gv'], F)) + list(_pack_gvl(params['centroid'], F)))
    ts, td, cs, cv8 = _node_pre(x_src, pos_src8, x_dst, pos_dst8, c1, w1)

    gd, gs = _sc_gather(td, ts, col, row)

    eg = msg1['edge_gvp']
    egWs, egwv, egb, egg = _pack_gvl(eg['gv'], 2 * EDGE_DIM)
    ec = _gvl_consts(eg['gv'])
    oWs, owv, ob, og = _pack_gvl(msg1['out_gv'], F)
    oc = _gvl_consts(msg1['out_gv'])
    c3 = jnp.stack(ec + [eg['vn_dir'][0, 0], params['vec_exp_w'][0, 0],
                         msg1['edge_vn']['w'][0, 0], msg1['e2n']['b'][0],
                         msg1['n2e']['b'][0]] + oc)
    w3 = [egWs[:EDGE_DIM], egWs[EDGE_DIM:], egwv, egb, egg,
          jnp.transpose(msg1['sca_linear']['w']), msg1['sca_linear']['b'][None, :],
          jnp.transpose(msg1['e2n']['w']), jnp.transpose(msg1['n2e']['w']),
          oWs, owv, ob, og]
    m, mv = _edge_stage(gd, gs, edge_attr.astype(f32), c3, w3)

    p = _sc_scatter(m, mv, row)

    ot = params['out_transform']
    c5 = jnp.stack(_gvl_consts(ot) + [params['act_vec_w'][0, 0]])
    w5 = [params['ln_sca']['g'][None, :], params['ln_sca']['b'][None, :],
          jnp.pad(params['ln_vec']['g'], ((0, 0), (0, 5))),
          jnp.pad(params['ln_vec']['b'], ((0, 0), (0, 5)))] + list(_pack_gvl(ot, F))
    out_s, out_v8 = _node_post(p, cs, cv8, c5, w5)

    return out_s, out_v8[:, :3].reshape(N, 1, 3)


# speedup vs baseline: 3.5311x; 1.0413x over previous
import functools
import math

import jax
import jax.numpy as jnp
from jax import lax
from jax.experimental import pallas as pl
from jax.experimental.pallas import tpu as pltpu
from jax.experimental.pallas import tpu_sc as plsc

N = 10000
E = 320000
F = 128
WG = 256
EDGE_DIM = 16
CUTOFF = 10.0
EPS = 1e-6

BN = 2000
BE = 2560

NC = 2
NS = 16
NW = NC * NS
PER_T = E // NW
CH = 80
NCH = PER_T // CH

SCH = 128
TOT_CHUNKS = E // SCH
TPS_CHUNKS = TOT_CHUNKS // NS
TPS_EXTRA = TOT_CHUNKS - NS * TPS_CHUNKS


def _leaky(x):
    return jnp.where(x >= 0, x, 0.01 * x)


def _elu(x):
    return jnp.where(x > 0, x, jnp.exp(jnp.minimum(x, 0.0)) - 1.0)


def _sum3(x):
    return jnp.dot(x, jnp.ones((3, 1), jnp.float32),
                   preferred_element_type=jnp.float32)


def _vn_leaky(w00, x):
    q = _sum3(x * x)
    dot = w00 * q
    dsq = w00 * w00 * q
    scale = 0.01 + 0.99 * jnp.where(dot >= 0, 1.0, EPS / (dsq + EPS))
    return x * scale


def _gvl(sca, vec, Ws_ref, wsv_ref, bs_ref, gw_ref, c5):
    lvw, lvb, l2w, l2b, gb = c5
    vi = lvw * vec + lvb
    vn = jnp.sqrt(jnp.sum(vi * vi, axis=1, keepdims=True) + 1e-12)
    so = (jnp.dot(sca, Ws_ref[...], preferred_element_type=jnp.float32)
          + vn * wsv_ref[...] + bs_ref[...])
    gate = jax.nn.sigmoid(
        jnp.dot(so, gw_ref[...], preferred_element_type=jnp.float32) + gb)
    return so, gate * (l2w * vi + l2b)


def _pack_gvl(p, in_s):
    w = p['lin_scalar']['w']
    return (jnp.transpose(w[:, :in_s]),
            jnp.transpose(w[:, in_s:in_s + 1]),
            p['lin_scalar']['b'][None, :],
            jnp.transpose(p['gates']['w']))


def _gvl_consts(p):
    return [p['lin_vector']['w'][0, 0], p['lin_vector']['b'][0],
            p['lin_vector2']['w'][0, 0], p['lin_vector2']['b'][0],
            p['gates']['b'][0]]


def _node_pre_body(consts, xs_ref, ps_ref, xd_ref, pd_ref,
                   s1Ws, s1wv, s1b, s1g, d1Ws, d1wv, d1b, d1g,
                   ngWs, ngwv, ngb, ngg, ctWs, ctwv, ctb, ctg,
                   Ts_ref, Td_ref, cs_ref, cv_ref):
    def c5(i):
        return tuple(consts[i + k] for k in range(5))

    for side in range(2):
        x = (xs_ref if side == 0 else xd_ref)[...]
        p3 = (ps_ref if side == 0 else pd_ref)[:, 0:3]
        pk = (s1Ws, s1wv, s1b, s1g) if side == 0 else (d1Ws, d1wv, d1b, d1g)
        so, vo = _gvl(x, p3, *pk, c5(0 if side == 0 else 5))
        xn = _leaky(so)
        pv = _vn_leaky(consts[10 + side], vo)
        ns, nv = _gvl(xn, pv, ngWs, ngwv, ngb, ngg, c5(12))
        T_ref = Ts_ref if side == 0 else Td_ref
        T_ref[:, 0:F] = ns
        T_ref[:, F:F + 3] = nv
        T_ref[:, F + 3:F + 6] = p3
        T_ref[:, F + 6:WG] = jnp.zeros((BN, WG - F - 6), jnp.float32)
        if side == 0:
            cs, cv = _gvl(xn, pv, ctWs, ctwv, ctb, ctg, c5(17))
            cs_ref[...] = cs
            cv_ref[:, 0:3] = cv
            cv_ref[:, 3:8] = jnp.zeros((BN, 5), jnp.float32)


def _node_pre(x_src, pos_src8, x_dst, pos_dst8, consts, wlist):
    full = lambda a: pl.BlockSpec(a.shape, lambda i: (0,) * a.ndim)
    blk = lambda s: pl.BlockSpec(s, lambda i: (i,) + (0,) * (len(s) - 1))
    return pl.pallas_call(
        _node_pre_body,
        grid=(N // BN,),
        in_specs=[pl.BlockSpec(memory_space=pltpu.SMEM),
                  blk((BN, F)), blk((BN, 8)), blk((BN, F)), blk((BN, 8))]
                 + [full(a) for a in wlist],
        out_specs=[blk((BN, WG)), blk((BN, WG)), blk((BN, F)), blk((BN, 8))],
        out_shape=[jax.ShapeDtypeStruct((N, WG), jnp.float32),
                   jax.ShapeDtypeStruct((N, WG), jnp.float32),
                   jax.ShapeDtypeStruct((N, F), jnp.float32),
                   jax.ShapeDtypeStruct((N, 8), jnp.float32)],
    )(consts, x_src, pos_src8, x_dst, pos_dst8, *wlist)


def _sc_gather(td, ts, col, row):
    mesh = plsc.VectorSubcoreMesh(core_axis_name="c", subcore_axis_name="s",
                                  num_cores=NC, num_subcores=NS)

    @functools.partial(
        pl.kernel, mesh=mesh,
        out_type=[jax.ShapeDtypeStruct((E, WG), jnp.float32),
                  jax.ShapeDtypeStruct((E, WG), jnp.float32)],
        scratch_types=[pltpu.VMEM((PER_T,), jnp.int32),
                       pltpu.VMEM((PER_T,), jnp.int32),
                       pltpu.VMEM((CH, WG), jnp.float32),
                       pltpu.VMEM((CH, WG), jnp.float32),
                       pltpu.VMEM((CH, WG), jnp.float32),
                       pltpu.VMEM((CH, WG), jnp.float32),
                       pltpu.SemaphoreType.DMA,
                       pltpu.SemaphoreType.DMA,
                       pltpu.SemaphoreType.DMA,
                       pltpu.SemaphoreType.DMA,
                       pltpu.SemaphoreType.DMA,
                       pltpu.SemaphoreType.DMA,
                       pltpu.SemaphoreType.DMA,
                       pltpu.SemaphoreType.DMA],
    )
    def gather_k(td_h, ts_h, col_h, row_h, gd_h, gs_h,
                 idx_d, idx_s, rows_d0, rows_s0, rows_d1, rows_s1,
                 sem_d0, sem_s0, sem_d1, sem_s1,
                 wsem_d0, wsem_s0, wsem_d1, wsem_s1):
        wid = lax.axis_index("s") * NC + lax.axis_index("c")
        base = wid * PER_T

        pltpu.sync_copy(col_h.at[pl.ds(base, PER_T)], idx_d)
        pltpu.sync_copy(row_h.at[pl.ds(base, PER_T)], idx_s)

        def drain(j2):
            off = base + (2 * j2 - 2) * CH
            pltpu.make_async_copy(rows_d0, gd_h.at[pl.ds(off, CH)], wsem_d0).wait()
            pltpu.make_async_copy(rows_s0, gs_h.at[pl.ds(off, CH)], wsem_s0).wait()
            off1 = off + CH
            pltpu.make_async_copy(rows_d1, gd_h.at[pl.ds(off1, CH)], wsem_d1).wait()
            pltpu.make_async_copy(rows_s1, gs_h.at[pl.ds(off1, CH)], wsem_s1).wait()

        def pair(j2, _):
            @pl.when(j2 > 0)
            def _():
                drain(j2)

            j0 = 2 * j2
            j1 = j0 + 1
            off0 = base + j0 * CH
            off1 = base + j1 * CH
            g0d = pltpu.async_copy(
                td_h.at[idx_d.at[pl.ds(j0 * CH, CH)]], rows_d0, sem_d0)
            g0s = pltpu.async_copy(
                ts_h.at[idx_s.at[pl.ds(j0 * CH, CH)]], rows_s0, sem_s0)
            g1d = pltpu.async_copy(
                td_h.at[idx_d.at[pl.ds(j1 * CH, CH)]], rows_d1, sem_d1)
            g1s = pltpu.async_copy(
                ts_h.at[idx_s.at[pl.ds(j1 * CH, CH)]], rows_s1, sem_s1)
            g0d.wait()
            pltpu.async_copy(rows_d0, gd_h.at[pl.ds(off0, CH)], wsem_d0)
            g0s.wait()
            pltpu.async_copy(rows_s0, gs_h.at[pl.ds(off0, CH)], wsem_s0)
            g1d.wait()
            pltpu.async_copy(rows_d1, gd_h.at[pl.ds(off1, CH)], wsem_d1)
            g1s.wait()
            pltpu.async_copy(rows_s1, gs_h.at[pl.ds(off1, CH)], wsem_s1)
            return 0

        lax.fori_loop(0, NCH // 2, pair, 0)
        drain(NCH // 2)

        j = NCH - 1
        off = base + j * CH
        gd_t = pltpu.async_copy(
            td_h.at[idx_d.at[pl.ds(j * CH, CH)]], rows_d0, sem_d0)
        gs_t = pltpu.async_copy(
            ts_h.at[idx_s.at[pl.ds(j * CH, CH)]], rows_s0, sem_s0)
        gd_t.wait()
        pltpu.sync_copy(rows_d0, gd_h.at[pl.ds(off, CH)])
        gs_t.wait()
        pltpu.sync_copy(rows_s0, gs_h.at[pl.ds(off, CH)])

    return gather_k(td, ts, col, row)


def _edge_body(consts, gd_ref, gs_ref, ea_ref,
               Wsm, Wea, wev, be, gew, Wt, bt, we2n, wn2e,
               Wo, wov, bo, gow, m_ref, mv_ref):
    (elvw, elvb, el2w, el2b, geb, vn_e, vexw, evnw, be2n, bn2e,
     olvw, olvb, ol2w, ol2b, gob) = (consts[k] for k in range(15))

    pd = gd_ref[:, F + 3:F + 6]
    ps = gs_ref[:, F + 3:F + 6]
    ev = ps - pd
    d2 = _sum3(ev * ev)
    ed = jnp.sqrt(d2 + 1e-12)

    off = (lax.broadcasted_iota(jnp.int32, (1, EDGE_DIM), 1).astype(jnp.float32)
           * (CUTOFF / (EDGE_DIM - 1)))
    dd = ed - off
    sm = jnp.exp((-0.5 * (EDGE_DIM - 1) * (EDGE_DIM - 1) / (CUTOFF * CUTOFF)) * dd * dd)

    e_vec = ev * ((1.0 / (ed + 1e-7)) * vexw)

    vi_e = elvw * e_vec + elvb
    vne = jnp.sqrt(_sum3(vi_e * vi_e) + 1e-12)
    es0 = (jnp.dot(sm, Wsm[...], preferred_element_type=jnp.float32)
           + jnp.dot(ea_ref[...], Wea[...], preferred_element_type=jnp.float32)
           + vne * wev[...] + be[...])
    gate_e = jax.nn.sigmoid(
        jnp.dot(es0, gew[...], preferred_element_type=jnp.float32) + geb)
    ve = gate_e * (el2w * vi_e + el2b)
    es = _leaky(es0)
    evg = _vn_leaky(vn_e, ve)

    t = jnp.dot(es, Wt[...], preferred_element_type=jnp.float32) + bt[...]
    c1 = jnp.dot(es, we2n[...], preferred_element_type=jnp.float32) + be2n
    ev2 = evnw * evg

    u = d2 * ((math.pi / CUTOFF) ** 2)
    fact = [1.0]
    for k in range(1, 13):
        fact.append(fact[-1] * (2 * k - 1) * (2 * k))
    poly = ((-1.0) ** 12) / fact[12]
    for k in range(11, -1, -1):
        poly = poly * u + ((-1.0) ** k) / fact[k]
    C = 0.5 * (1.0 + poly)
    C = C * (ed <= CUTOFF).astype(jnp.float32) * (ed >= 0.0).astype(jnp.float32)

    def msg(ns, nv):
        y_sca = ns * t
        c2 = jnp.dot(ns, wn2e[...], preferred_element_type=jnp.float32) + bn2e
        y_v = c1 * nv + c2 * ev2
        vi = olvw * y_v + olvb
        vno = jnp.sqrt(_sum3(vi * vi) + 1e-12)
        os_ = (jnp.dot(y_sca, Wo[...], preferred_element_type=jnp.float32)
               + vno * wov[...] + bo[...])
        gate = jax.nn.sigmoid(
            jnp.dot(os_, gow[...], preferred_element_type=jnp.float32) + gob)
        ov = gate * (ol2w * vi + ol2b)
        return os_ * C, ov * C

    os1, ov1 = msg(gd_ref[:, 0:F], gd_ref[:, F:F + 3])
    os2, ov2 = msg(gs_ref[:, 0:F], gs_ref[:, F:F + 3])
    m_ref[...] = _elu((os1 + os2) * 0.5)
    mv_ref[:, 0:3] = _elu((ov1 + ov2) * 0.5)
    mv_ref[:, 3:F] = jnp.zeros((BE, F - 3), jnp.float32)


def _edge_stage(gd, gs, ea, consts, wlist):
    full = lambda a: pl.BlockSpec(a.shape, lambda i: (0,) * a.ndim)
    blk = lambda s: pl.BlockSpec(s, lambda i: (i,) + (0,) * (len(s) - 1))
    return pl.pallas_call(
        _edge_body,
        grid=(E // BE,),
        in_specs=[pl.BlockSpec(memory_space=pltpu.SMEM),
                  blk((BE, WG)), blk((BE, WG)), blk((BE, EDGE_DIM))]
                 + [full(a) for a in wlist],
        out_specs=[blk((BE, F)), blk((BE, F))],
        out_shape=[jax.ShapeDtypeStruct((E, F), jnp.float32),
                   jax.ShapeDtypeStruct((E, F), jnp.float32)],
    )(consts, gd, gs, ea, *wlist)


def _sc_scatter(m, mv, row):
    mesh = plsc.VectorSubcoreMesh(core_axis_name="c", subcore_axis_name="s",
                                  num_cores=NC, num_subcores=NS)

    @functools.partial(
        pl.kernel, mesh=mesh,
        out_type=jax.ShapeDtypeStruct((NC, N, F), jnp.float32),
        scratch_types=[pltpu.VMEM((SCH,), jnp.int32),
                       pltpu.VMEM((SCH,), jnp.int32),
                       pltpu.VMEM((SCH, F), jnp.float32),
                       pltpu.VMEM((SCH, F), jnp.float32),
                       pltpu.VMEM((8, F), jnp.float32),
                       pltpu.VMEM_SHARED((N, F), jnp.float32),
                       pltpu.SemaphoreType.DMA,
                       pltpu.SemaphoreType.DMA],
    )
    def scatter_k(m_h, mv_h, row_h, p_h,
                  idx_v0, idx_v1, mbuf0, mbuf1, zbuf, acc, sml0, sml1):
        c = lax.axis_index("c")
        s = lax.axis_index("s")

        zv16 = jnp.zeros((16,), jnp.float32)

        def zrow(i, _):
            for k in range(F // 16):
                zbuf[i, pl.ds(16 * k, 16)] = zv16
            return 0

        lax.fori_loop(0, 8, zrow, 0)

        nrc = 78 + jnp.where(s < 2, 1, 0)
        rbase = s * 624 + 8 * jnp.minimum(s, 2)

        def za(j, _):
            pltpu.sync_copy(zbuf, acc.at[pl.ds(rbase + 8 * j, 8)])
            return 0

        lax.fori_loop(0, nrc, za, 0)
        plsc.subcore_barrier()

        nch = TPS_CHUNKS + jnp.where(s < TPS_EXTRA, 1, 0)
        cbase = s * TPS_CHUNKS + jnp.minimum(s, TPS_EXTRA)

        def mkloop(src_ref):
            def pair(j2, _):
                j0 = 2 * j2
                j1 = j0 + 1
                j1c = jnp.minimum(j1, nch - 1)
                off0 = (cbase + j0) * SCH
                off1 = (cbase + j1c) * SCH
                l0 = pltpu.async_copy(src_ref.at[pl.ds(off0, SCH)], mbuf0, sml0)
                l1 = pltpu.async_copy(src_ref.at[pl.ds(off1, SCH)], mbuf1, sml1)
                pltpu.sync_copy(row_h.at[pl.ds(off0, SCH)], idx_v0)
                pltpu.sync_copy(row_h.at[pl.ds(off1, SCH)], idx_v1)
                l0.wait()
                pltpu.sync_copy(mbuf0, acc.at[idx_v0], add=True)
                l1.wait()

                @pl.when(j1 < nch)
                def _():
                    pltpu.sync_copy(mbuf1, acc.at[idx_v1], add=True)

                return 0
            return pair

        npairs = (nch + 1) // 2

        @pl.when(c == 0)
        def _():
            lax.fori_loop(0, npairs, mkloop(m_h), 0)

        @pl.when(c == 1)
        def _():
            lax.fori_loop(0, npairs, mkloop(mv_h), 0)

        plsc.subcore_barrier()

        def wb(j, _):
            r = rbase + 8 * j
            pltpu.sync_copy(acc.at[pl.ds(r, 8)], zbuf)
            pltpu.sync_copy(zbuf, p_h.at[c, pl.ds(r, 8)])
            return 0

        lax.fori_loop(0, nrc, wb, 0)

    return scatter_k(m, mv, row)


def _post_body(consts, p_ref, cs_ref, cv_ref,
               lng, lnb, lvg, lvb_, tWs, twv, tb, tg, out_s_ref, out_v_ref):
    tlvw, tlvb, tl2w, tl2b, tgb, actw = (consts[k] for k in range(6))
    s = cs_ref[...] + p_ref[0, :, :]
    v = cv_ref[:, 0:3] + p_ref[1, :, 0:3]
    m = jnp.mean(s, axis=1, keepdims=True)
    va = jnp.mean((s - m) * (s - m), axis=1, keepdims=True)
    s = (s - m) / jnp.sqrt(va + 1e-5) * lng[...] + lnb[...]
    mv = jnp.mean(v, axis=1, keepdims=True)
    vv = jnp.mean((v - mv) * (v - mv), axis=1, keepdims=True)
    v = (v - mv) / jnp.sqrt(vv + 1e-5) * lvg[:, 0:3] + lvb_[:, 0:3]
    s = _leaky(s)
    v = _vn_leaky(actw, v)
    so, vo = _gvl(s, v, tWs, twv, tb, tg, (tlvw, tlvb, tl2w, tl2b, tgb))
    out_s_ref[...] = so
    out_v_ref[:, 0:3] = vo
    out_v_ref[:, 3:8] = jnp.zeros((BN, 5), jnp.float32)


def _node_post(p, cs, cv8, consts, wlist):
    full = lambda a: pl.BlockSpec(a.shape, lambda i: (0,) * a.ndim)
    blk = lambda s: pl.BlockSpec(s, lambda i: (i,) + (0,) * (len(s) - 1))
    return pl.pallas_call(
        _post_body,
        grid=(N // BN,),
        in_specs=[pl.BlockSpec(memory_space=pltpu.SMEM),
                  pl.BlockSpec((NC, BN, F), lambda i: (0, i, 0)),
                  blk((BN, F)), blk((BN, 8))]
                 + [full(a) for a in wlist],
        out_specs=[blk((BN, F)), blk((BN, 8))],
        out_shape=[jax.ShapeDtypeStruct((N, F), jnp.float32),
                   jax.ShapeDtypeStruct((N, 8), jnp.float32)],
    )(consts, p, cs, cv8, *wlist)


def kernel(x_src, pos_src, x_dst, pos_dst, edge_index, edge_attr, params):
    f32 = jnp.float32
    row = edge_index[0]
    col = edge_index[1]
    pos_src8 = jnp.pad(pos_src.astype(f32), ((0, 0), (0, 5)))
    pos_dst8 = jnp.pad(pos_dst.astype(f32), ((0, 0), (0, 5)))

    p1s, p1d = params['per1_src'], params['per1_dst']
    msg1 = params['msg1']
    c1 = jnp.stack(
        _gvl_consts(p1s['gv']) + _gvl_consts(p1d['gv'])
        + [p1s['vn_dir'][0, 0], p1d['vn_dir'][0, 0]]
        + _gvl_consts(msg1['node_gv']) + _gvl_consts(params['centroid']))
    w1 = (list(_pack_gvl(p1s['gv'], F)) + list(_pack_gvl(p1d['gv'], F))
          + list(_pack_gvl(msg1['node_gv'], F)) + list(_pack_gvl(params['centroid'], F)))
    ts, td, cs, cv8 = _node_pre(x_src, pos_src8, x_dst, pos_dst8, c1, w1)

    gd, gs = _sc_gather(td, ts, col, row)

    eg = msg1['edge_gvp']
    egWs, egwv, egb, egg = _pack_gvl(eg['gv'], 2 * EDGE_DIM)
    ec = _gvl_consts(eg['gv'])
    oWs, owv, ob, og = _pack_gvl(msg1['out_gv'], F)
    oc = _gvl_consts(msg1['out_gv'])
    c3 = jnp.stack(ec + [eg['vn_dir'][0, 0], params['vec_exp_w'][0, 0],
                         msg1['edge_vn']['w'][0, 0], msg1['e2n']['b'][0],
                         msg1['n2e']['b'][0]] + oc)
    w3 = [egWs[:EDGE_DIM], egWs[EDGE_DIM:], egwv, egb, egg,
          jnp.transpose(msg1['sca_linear']['w']), msg1['sca_linear']['b'][None, :],
          jnp.transpose(msg1['e2n']['w']), jnp.transpose(msg1['n2e']['w']),
          oWs, owv, ob, og]
    m, mv = _edge_stage(gd, gs, edge_attr.astype(f32), c3, w3)

    p = _sc_scatter(m, mv, row)

    ot = params['out_transform']
    c5 = jnp.stack(_gvl_consts(ot) + [params['act_vec_w'][0, 0]])
    w5 = [params['ln_sca']['g'][None, :], params['ln_sca']['b'][None, :],
          jnp.pad(params['ln_vec']['g'], ((0, 0), (0, 5))),
          jnp.pad(params['ln_vec']['b'], ((0, 0), (0, 5)))] + list(_pack_gvl(ot, F))
    out_s, out_v8 = _node_post(p, cs, cv8, c5, w5)

    return out_s, out_v8[:, :3].reshape(N, 1, 3)


# speedup vs baseline: 3.9463x; 1.1176x over previous
import functools
import math

import jax
import jax.numpy as jnp
from jax import lax
from jax.experimental import pallas as pl
from jax.experimental.pallas import tpu as pltpu
from jax.experimental.pallas import tpu_sc as plsc

N = 10000
E = 320000
F = 128
WG = 256
EDGE_DIM = 16
CUTOFF = 10.0
EPS = 1e-6

BN = 2000
BE = 2000

NC = 2
NS = 16
NW = NC * NS
PER_T = E // NW
CH = 80
NCH = PER_T // CH

SCH = 128
TOT_CHUNKS = E // SCH
TPS_CHUNKS = TOT_CHUNKS // NS
TPS_EXTRA = TOT_CHUNKS - NS * TPS_CHUNKS


def _leaky(x):
    return jnp.where(x >= 0, x, 0.01 * x)


def _elu(x):
    return jnp.where(x > 0, x, jnp.exp(jnp.minimum(x, 0.0)) - 1.0)


def _sum3(x):
    return jnp.dot(x, jnp.ones((3, 1), jnp.float32),
                   preferred_element_type=jnp.float32)


def _vn_leaky(w00, x):
    q = _sum3(x * x)
    dot = w00 * q
    dsq = w00 * w00 * q
    scale = 0.01 + 0.99 * jnp.where(dot >= 0, 1.0, EPS / (dsq + EPS))
    return x * scale


def _gvl(sca, vec, Ws_ref, wsv_ref, bs_ref, gw_ref, c5):
    lvw, lvb, l2w, l2b, gb = c5
    vi = lvw * vec + lvb
    vn = jnp.sqrt(jnp.sum(vi * vi, axis=1, keepdims=True) + 1e-12)
    so = (jnp.dot(sca, Ws_ref[...], preferred_element_type=jnp.float32)
          + vn * wsv_ref[...] + bs_ref[...])
    gate = jax.nn.sigmoid(
        jnp.dot(so, gw_ref[...], preferred_element_type=jnp.float32) + gb)
    return so, gate * (l2w * vi + l2b)


def _pack_gvl(p, in_s):
    w = p['lin_scalar']['w']
    return (jnp.transpose(w[:, :in_s]),
            jnp.transpose(w[:, in_s:in_s + 1]),
            p['lin_scalar']['b'][None, :],
            jnp.transpose(p['gates']['w']))


def _gvl_consts(p):
    return [p['lin_vector']['w'][0, 0], p['lin_vector']['b'][0],
            p['lin_vector2']['w'][0, 0], p['lin_vector2']['b'][0],
            p['gates']['b'][0]]


def _node_pre_body(consts, xs_ref, ps_ref, xd_ref, pd_ref,
                   s1Ws, s1wv, s1b, s1g, d1Ws, d1wv, d1b, d1g,
                   ngWs, ngwv, ngb, ngg, ctWs, ctwv, ctb, ctg,
                   Ts_ref, Td_ref, cs_ref, cv_ref):
    def c5(i):
        return tuple(consts[i + k] for k in range(5))

    for side in range(2):
        x = (xs_ref if side == 0 else xd_ref)[...]
        p3 = (ps_ref if side == 0 else pd_ref)[:, 0:3]
        pk = (s1Ws, s1wv, s1b, s1g) if side == 0 else (d1Ws, d1wv, d1b, d1g)
        so, vo = _gvl(x, p3, *pk, c5(0 if side == 0 else 5))
        xn = _leaky(so)
        pv = _vn_leaky(consts[10 + side], vo)
        ns, nv = _gvl(xn, pv, ngWs, ngwv, ngb, ngg, c5(12))
        T_ref = Ts_ref if side == 0 else Td_ref
        T_ref[:, 0:F] = ns
        T_ref[:, F:F + 3] = nv
        T_ref[:, F + 3:F + 6] = p3
        T_ref[:, F + 6:WG] = jnp.zeros((BN, WG - F - 6), jnp.float32)
        if side == 0:
            cs, cv = _gvl(xn, pv, ctWs, ctwv, ctb, ctg, c5(17))
            cs_ref[...] = cs
            cv_ref[:, 0:3] = cv
            cv_ref[:, 3:8] = jnp.zeros((BN, 5), jnp.float32)


def _node_pre(x_src, pos_src8, x_dst, pos_dst8, consts, wlist):
    full = lambda a: pl.BlockSpec(a.shape, lambda i: (0,) * a.ndim)
    blk = lambda s: pl.BlockSpec(s, lambda i: (i,) + (0,) * (len(s) - 1))
    return pl.pallas_call(
        _node_pre_body,
        grid=(N // BN,),
        in_specs=[pl.BlockSpec(memory_space=pltpu.SMEM),
                  blk((BN, F)), blk((BN, 8)), blk((BN, F)), blk((BN, 8))]
                 + [full(a) for a in wlist],
        out_specs=[blk((BN, WG)), blk((BN, WG)), blk((BN, F)), blk((BN, 8))],
        out_shape=[jax.ShapeDtypeStruct((N, WG), jnp.float32),
                   jax.ShapeDtypeStruct((N, WG), jnp.float32),
                   jax.ShapeDtypeStruct((N, F), jnp.float32),
                   jax.ShapeDtypeStruct((N, 8), jnp.float32)],
    )(consts, x_src, pos_src8, x_dst, pos_dst8, *wlist)


def _sc_gather(td, ts, col, row):
    eh = col.shape[0]
    per_t = eh // NW
    ch = CH if per_t % (2 * CH) == 0 else CH // 2
    nch = per_t // ch
    mesh = plsc.VectorSubcoreMesh(core_axis_name="c", subcore_axis_name="s",
                                  num_cores=NC, num_subcores=NS)

    @functools.partial(
        pl.kernel, mesh=mesh,
        out_type=[jax.ShapeDtypeStruct((eh, WG), jnp.float32),
                  jax.ShapeDtypeStruct((eh, WG), jnp.float32)],
        scratch_types=[pltpu.VMEM((per_t,), jnp.int32),
                       pltpu.VMEM((per_t,), jnp.int32),
                       pltpu.VMEM((ch, WG), jnp.float32),
                       pltpu.VMEM((ch, WG), jnp.float32),
                       pltpu.VMEM((ch, WG), jnp.float32),
                       pltpu.VMEM((ch, WG), jnp.float32),
                       pltpu.SemaphoreType.DMA,
                       pltpu.SemaphoreType.DMA,
                       pltpu.SemaphoreType.DMA,
                       pltpu.SemaphoreType.DMA,
                       pltpu.SemaphoreType.DMA,
                       pltpu.SemaphoreType.DMA,
                       pltpu.SemaphoreType.DMA,
                       pltpu.SemaphoreType.DMA],
    )
    def gather_k(td_h, ts_h, col_h, row_h, gd_h, gs_h,
                 idx_d, idx_s, rows_d0, rows_s0, rows_d1, rows_s1,
                 sem_d0, sem_s0, sem_d1, sem_s1,
                 wsem_d0, wsem_s0, wsem_d1, wsem_s1):
        wid = lax.axis_index("s") * NC + lax.axis_index("c")
        base = wid * per_t

        pltpu.sync_copy(col_h.at[pl.ds(base, per_t)], idx_d)
        pltpu.sync_copy(row_h.at[pl.ds(base, per_t)], idx_s)

        def drain(j2):
            off = base + (2 * j2 - 2) * ch
            pltpu.make_async_copy(rows_d0, gd_h.at[pl.ds(off, ch)], wsem_d0).wait()
            pltpu.make_async_copy(rows_s0, gs_h.at[pl.ds(off, ch)], wsem_s0).wait()
            off1 = off + CH
            pltpu.make_async_copy(rows_d1, gd_h.at[pl.ds(off1, ch)], wsem_d1).wait()
            pltpu.make_async_copy(rows_s1, gs_h.at[pl.ds(off1, ch)], wsem_s1).wait()

        def pair(j2, _):
            @pl.when(j2 > 0)
            def _():
                drain(j2)

            j0 = 2 * j2
            j1 = j0 + 1
            off0 = base + j0 * ch
            off1 = base + j1 * ch
            g0d = pltpu.async_copy(
                td_h.at[idx_d.at[pl.ds(j0 * ch, ch)]], rows_d0, sem_d0)
            g0s = pltpu.async_copy(
                ts_h.at[idx_s.at[pl.ds(j0 * ch, ch)]], rows_s0, sem_s0)
            g1d = pltpu.async_copy(
                td_h.at[idx_d.at[pl.ds(j1 * ch, ch)]], rows_d1, sem_d1)
            g1s = pltpu.async_copy(
                ts_h.at[idx_s.at[pl.ds(j1 * ch, ch)]], rows_s1, sem_s1)
            g0d.wait()
            pltpu.async_copy(rows_d0, gd_h.at[pl.ds(off0, ch)], wsem_d0)
            g0s.wait()
            pltpu.async_copy(rows_s0, gs_h.at[pl.ds(off0, ch)], wsem_s0)
            g1d.wait()
            pltpu.async_copy(rows_d1, gd_h.at[pl.ds(off1, ch)], wsem_d1)
            g1s.wait()
            pltpu.async_copy(rows_s1, gs_h.at[pl.ds(off1, ch)], wsem_s1)
            return 0

        lax.fori_loop(0, nch // 2, pair, 0)
        drain(nch // 2)

        j = nch - 1
        off = base + j * ch
        gd_t = pltpu.async_copy(
            td_h.at[idx_d.at[pl.ds(j * ch, ch)]], rows_d0, sem_d0)
        gs_t = pltpu.async_copy(
            ts_h.at[idx_s.at[pl.ds(j * ch, ch)]], rows_s0, sem_s0)
        gd_t.wait()
        pltpu.sync_copy(rows_d0, gd_h.at[pl.ds(off, ch)])
        gs_t.wait()
        pltpu.sync_copy(rows_s0, gs_h.at[pl.ds(off, ch)])

    return gather_k(td, ts, col, row)


def _edge_body(consts, gd_ref, gs_ref, ea_ref,
               Wsm, Wea, wev, be, gew, Wt, bt, we2n, wn2e,
               Wo, wov, bo, gow, m_ref, mv_ref):
    (elvw, elvb, el2w, el2b, geb, vn_e, vexw, evnw, be2n, bn2e,
     olvw, olvb, ol2w, ol2b, gob) = (consts[k] for k in range(15))

    pd = gd_ref[:, F + 3:F + 6]
    ps = gs_ref[:, F + 3:F + 6]
    ev = ps - pd
    d2 = _sum3(ev * ev)
    ed = jnp.sqrt(d2 + 1e-12)

    off = (lax.broadcasted_iota(jnp.int32, (1, EDGE_DIM), 1).astype(jnp.float32)
           * (CUTOFF / (EDGE_DIM - 1)))
    dd = ed - off
    sm = jnp.exp((-0.5 * (EDGE_DIM - 1) * (EDGE_DIM - 1) / (CUTOFF * CUTOFF)) * dd * dd)

    e_vec = ev * ((1.0 / (ed + 1e-7)) * vexw)

    vi_e = elvw * e_vec + elvb
    vne = jnp.sqrt(_sum3(vi_e * vi_e) + 1e-12)
    es0 = (jnp.dot(sm, Wsm[...], preferred_element_type=jnp.float32)
           + jnp.dot(ea_ref[...], Wea[...], preferred_element_type=jnp.float32)
           + vne * wev[...] + be[...])
    gate_e = jax.nn.sigmoid(
        jnp.dot(es0, gew[...], preferred_element_type=jnp.float32) + geb)
    ve = gate_e * (el2w * vi_e + el2b)
    es = _leaky(es0)
    evg = _vn_leaky(vn_e, ve)

    t = jnp.dot(es, Wt[...], preferred_element_type=jnp.float32) + bt[...]
    c1 = jnp.dot(es, we2n[...], preferred_element_type=jnp.float32) + be2n
    ev2 = evnw * evg

    u = d2 * ((math.pi / CUTOFF) ** 2)
    fact = [1.0]
    for k in range(1, 13):
        fact.append(fact[-1] * (2 * k - 1) * (2 * k))
    poly = ((-1.0) ** 12) / fact[12]
    for k in range(11, -1, -1):
        poly = poly * u + ((-1.0) ** k) / fact[k]
    C = 0.5 * (1.0 + poly)
    C = C * (ed <= CUTOFF).astype(jnp.float32) * (ed >= 0.0).astype(jnp.float32)

    def msg(ns, nv):
        y_sca = ns * t
        c2 = jnp.dot(ns, wn2e[...], preferred_element_type=jnp.float32) + bn2e
        y_v = c1 * nv + c2 * ev2
        vi = olvw * y_v + olvb
        vno = jnp.sqrt(_sum3(vi * vi) + 1e-12)
        os_ = (jnp.dot(y_sca, Wo[...], preferred_element_type=jnp.float32)
               + vno * wov[...] + bo[...])
        gate = jax.nn.sigmoid(
            jnp.dot(os_, gow[...], preferred_element_type=jnp.float32) + gob)
        ov = gate * (ol2w * vi + ol2b)
        return os_ * C, ov * C

    os1, ov1 = msg(gd_ref[:, 0:F], gd_ref[:, F:F + 3])
    os2, ov2 = msg(gs_ref[:, 0:F], gs_ref[:, F:F + 3])
    m_ref[...] = _elu((os1 + os2) * 0.5)
    mv_ref[:, 0:3] = _elu((ov1 + ov2) * 0.5)
    mv_ref[:, 3:F] = jnp.zeros((BE, F - 3), jnp.float32)


def _edge_stage(gd, gs, ea, consts, wlist):
    full = lambda a: pl.BlockSpec(a.shape, lambda i: (0,) * a.ndim)
    blk = lambda s: pl.BlockSpec(s, lambda i: (i,) + (0,) * (len(s) - 1))
    return pl.pallas_call(
        _edge_body,
        grid=(gd.shape[0] // BE,),
        in_specs=[pl.BlockSpec(memory_space=pltpu.SMEM),
                  blk((BE, WG)), blk((BE, WG)), blk((BE, EDGE_DIM))]
                 + [full(a) for a in wlist],
        out_specs=[blk((BE, F)), blk((BE, F))],
        out_shape=[jax.ShapeDtypeStruct((gd.shape[0], F), jnp.float32),
                   jax.ShapeDtypeStruct((gd.shape[0], F), jnp.float32)],
    )(consts, gd, gs, ea, *wlist)


def _sc_scatter(m1, m2, mv1, mv2, row):
    mesh = plsc.VectorSubcoreMesh(core_axis_name="c", subcore_axis_name="s",
                                  num_cores=NC, num_subcores=NS)

    HTOT = (E // 2) // SCH
    HTPS = HTOT // 8
    HEXT = HTOT - 8 * HTPS

    @functools.partial(
        pl.kernel, mesh=mesh,
        out_type=jax.ShapeDtypeStruct((NC, N, F), jnp.float32),
        scratch_types=[pltpu.VMEM((SCH,), jnp.int32),
                       pltpu.VMEM((SCH,), jnp.int32),
                       pltpu.VMEM((SCH, F), jnp.float32),
                       pltpu.VMEM((SCH, F), jnp.float32),
                       pltpu.VMEM((8, F), jnp.float32),
                       pltpu.VMEM_SHARED((N, F), jnp.float32),
                       pltpu.SemaphoreType.DMA,
                       pltpu.SemaphoreType.DMA],
    )
    def scatter_k(m1_h, m2_h, mv1_h, mv2_h, row_h, p_h,
                  idx_v0, idx_v1, mbuf0, mbuf1, zbuf, acc, sml0, sml1):
        c = lax.axis_index("c")
        s = lax.axis_index("s")

        zv16 = jnp.zeros((16,), jnp.float32)

        def zrow(i, _):
            for k in range(F // 16):
                zbuf[i, pl.ds(16 * k, 16)] = zv16
            return 0

        lax.fori_loop(0, 8, zrow, 0)

        nrc = 78 + jnp.where(s < 2, 1, 0)
        rbase = s * 624 + 8 * jnp.minimum(s, 2)

        def za(j, _):
            pltpu.sync_copy(zbuf, acc.at[pl.ds(rbase + 8 * j, 8)])
            return 0

        lax.fori_loop(0, nrc, za, 0)
        plsc.subcore_barrier()

        s8 = lax.rem(s, 8)
        nch = HTPS + jnp.where(s8 < HEXT, 1, 0)
        cbase = s8 * HTPS + jnp.minimum(s8, HEXT)
        npairs = (nch + 1) // 2

        def mkloop(src_ref, ebase):
            def pair(j2, _):
                j0 = 2 * j2
                j1 = j0 + 1
                j1c = jnp.minimum(j1, nch - 1)
                off0 = (cbase + j0) * SCH
                off1 = (cbase + j1c) * SCH
                l0 = pltpu.async_copy(src_ref.at[pl.ds(off0, SCH)], mbuf0, sml0)
                l1 = pltpu.async_copy(src_ref.at[pl.ds(off1, SCH)], mbuf1, sml1)
                pltpu.sync_copy(row_h.at[pl.ds(ebase + off0, SCH)], idx_v0)
                pltpu.sync_copy(row_h.at[pl.ds(ebase + off1, SCH)], idx_v1)
                l0.wait()
                pltpu.sync_copy(mbuf0, acc.at[idx_v0], add=True)
                l1.wait()

                @pl.when(j1 < nch)
                def _():
                    pltpu.sync_copy(mbuf1, acc.at[idx_v1], add=True)

                return 0
            return pair

        @pl.when(jnp.logical_and(c == 0, s < 8))
        def _():
            lax.fori_loop(0, npairs, mkloop(m1_h, 0), 0)

        @pl.when(jnp.logical_and(c == 0, s >= 8))
        def _():
            lax.fori_loop(0, npairs, mkloop(m2_h, E // 2), 0)

        @pl.when(jnp.logical_and(c == 1, s < 8))
        def _():
            lax.fori_loop(0, npairs, mkloop(mv1_h, 0), 0)

        @pl.when(jnp.logical_and(c == 1, s >= 8))
        def _():
            lax.fori_loop(0, npairs, mkloop(mv2_h, E // 2), 0)

        plsc.subcore_barrier()

        def wb(j, _):
            r = rbase + 8 * j
            pltpu.sync_copy(acc.at[pl.ds(r, 8)], zbuf)
            pltpu.sync_copy(zbuf, p_h.at[c, pl.ds(r, 8)])
            return 0

        lax.fori_loop(0, nrc, wb, 0)

    return scatter_k(m1, m2, mv1, mv2, row)


def _post_body(consts, p_ref, cs_ref, cv_ref,
               lng, lnb, lvg, lvb_, tWs, twv, tb, tg, out_s_ref, out_v_ref):
    tlvw, tlvb, tl2w, tl2b, tgb, actw = (consts[k] for k in range(6))
    s = cs_ref[...] + p_ref[0, :, :]
    v = cv_ref[:, 0:3] + p_ref[1, :, 0:3]
    m = jnp.mean(s, axis=1, keepdims=True)
    va = jnp.mean((s - m) * (s - m), axis=1, keepdims=True)
    s = (s - m) / jnp.sqrt(va + 1e-5) * lng[...] + lnb[...]
    mv = jnp.mean(v, axis=1, keepdims=True)
    vv = jnp.mean((v - mv) * (v - mv), axis=1, keepdims=True)
    v = (v - mv) / jnp.sqrt(vv + 1e-5) * lvg[:, 0:3] + lvb_[:, 0:3]
    s = _leaky(s)
    v = _vn_leaky(actw, v)
    so, vo = _gvl(s, v, tWs, twv, tb, tg, (tlvw, tlvb, tl2w, tl2b, tgb))
    out_s_ref[...] = so
    out_v_ref[:, 0:3] = vo
    out_v_ref[:, 3:8] = jnp.zeros((BN, 5), jnp.float32)


def _node_post(p, cs, cv8, consts, wlist):
    full = lambda a: pl.BlockSpec(a.shape, lambda i: (0,) * a.ndim)
    blk = lambda s: pl.BlockSpec(s, lambda i: (i,) + (0,) * (len(s) - 1))
    return pl.pallas_call(
        _post_body,
        grid=(N // BN,),
        in_specs=[pl.BlockSpec(memory_space=pltpu.SMEM),
                  pl.BlockSpec((NC, BN, F), lambda i: (0, i, 0)),
                  blk((BN, F)), blk((BN, 8))]
                 + [full(a) for a in wlist],
        out_specs=[blk((BN, F)), blk((BN, 8))],
        out_shape=[jax.ShapeDtypeStruct((N, F), jnp.float32),
                   jax.ShapeDtypeStruct((N, 8), jnp.float32)],
    )(consts, p, cs, cv8, *wlist)


def kernel(x_src, pos_src, x_dst, pos_dst, edge_index, edge_attr, params):
    f32 = jnp.float32
    row = edge_index[0]
    col = edge_index[1]
    pos_src8 = jnp.pad(pos_src.astype(f32), ((0, 0), (0, 5)))
    pos_dst8 = jnp.pad(pos_dst.astype(f32), ((0, 0), (0, 5)))

    p1s, p1d = params['per1_src'], params['per1_dst']
    msg1 = params['msg1']
    c1 = jnp.stack(
        _gvl_consts(p1s['gv']) + _gvl_consts(p1d['gv'])
        + [p1s['vn_dir'][0, 0], p1d['vn_dir'][0, 0]]
        + _gvl_consts(msg1['node_gv']) + _gvl_consts(params['centroid']))
    w1 = (list(_pack_gvl(p1s['gv'], F)) + list(_pack_gvl(p1d['gv'], F))
          + list(_pack_gvl(msg1['node_gv'], F)) + list(_pack_gvl(params['centroid'], F)))
    ts, td, cs, cv8 = _node_pre(x_src, pos_src8, x_dst, pos_dst8, c1, w1)

    h = E // 2
    gd1, gs1 = _sc_gather(td, ts, col[:h], row[:h])
    gd2, gs2 = _sc_gather(td, ts, col[h:], row[h:])

    eg = msg1['edge_gvp']
    egWs, egwv, egb, egg = _pack_gvl(eg['gv'], 2 * EDGE_DIM)
    ec = _gvl_consts(eg['gv'])
    oWs, owv, ob, og = _pack_gvl(msg1['out_gv'], F)
    oc = _gvl_consts(msg1['out_gv'])
    c3 = jnp.stack(ec + [eg['vn_dir'][0, 0], params['vec_exp_w'][0, 0],
                         msg1['edge_vn']['w'][0, 0], msg1['e2n']['b'][0],
                         msg1['n2e']['b'][0]] + oc)
    w3 = [egWs[:EDGE_DIM], egWs[EDGE_DIM:], egwv, egb, egg,
          jnp.transpose(msg1['sca_linear']['w']), msg1['sca_linear']['b'][None, :],
          jnp.transpose(msg1['e2n']['w']), jnp.transpose(msg1['n2e']['w']),
          oWs, owv, ob, og]
    m1, mv1 = _edge_stage(gd1, gs1, edge_attr[:h].astype(f32), c3, w3)
    m2, mv2 = _edge_stage(gd2, gs2, edge_attr[h:].astype(f32), c3, w3)

    p = _sc_scatter(m1, m2, mv1, mv2, row)

    ot = params['out_transform']
    c5 = jnp.stack(_gvl_consts(ot) + [params['act_vec_w'][0, 0]])
    w5 = [params['ln_sca']['g'][None, :], params['ln_sca']['b'][None, :],
          jnp.pad(params['ln_vec']['g'], ((0, 0), (0, 5))),
          jnp.pad(params['ln_vec']['b'], ((0, 0), (0, 5)))] + list(_pack_gvl(ot, F))
    out_s, out_v8 = _node_post(p, cs, cv8, c5, w5)

    return out_s, out_v8[:, :3].reshape(N, 1, 3)
